# baseline (speedup 1.0000x reference)
import functools

import jax
import jax.numpy as jnp
from jax import lax
from jax.experimental import pallas as pl
from jax.experimental.pallas import tpu as pltpu
from jax.experimental.pallas import tpu_sc as plsc

N = 10000
E = 320000
D = 128
EA = 16
R = 4
H = 4
C = 128
REL = 8
ED = EA + REL
HC = H * C
FFN = 256

TE, ET = 160, 2000
TN, NT = 10, 1000
NC, NS = 2, 16
NW = NC * NS
EW = E // NW
CH = 40
NIT = EW // CH
NP = 10240
NPS = NP // NS

f32 = jnp.float32
i32 = jnp.int32

_mesh = plsc.VectorSubcoreMesh(core_axis_name="c", subcore_axis_name="s")


def _tables_body(h_ref, wl_ref, bl_ref, wr_ref, br_ref, we_ref, re_ref,
                 xl_ref, xr_ref):
    h = h_ref[...]
    xl_ref[0] = h @ wl_ref[0] + bl_ref[0]
    crel = re_ref[0] @ we_ref[0, EA:, :]
    xr_ref[0] = h @ wr_ref[0] + br_ref[0] + crel


def _make_tables(h, W_l, b_l, W_r, b_r, W_e, rel_emb):
    return pl.pallas_call(
        _tables_body,
        grid=(R, TN),
        in_specs=[
            pl.BlockSpec((NT, D), lambda r, t: (t, 0)),
            pl.BlockSpec((1, D, HC), lambda r, t: (r, 0, 0)),
            pl.BlockSpec((1, 1, HC), lambda r, t: (r, 0, 0)),
            pl.BlockSpec((1, D, HC), lambda r, t: (r, 0, 0)),
            pl.BlockSpec((1, 1, HC), lambda r, t: (r, 0, 0)),
            pl.BlockSpec((1, ED, HC), lambda r, t: (r, 0, 0)),
            pl.BlockSpec((1, 1, REL), lambda r, t: (r, 0, 0)),
        ],
        out_specs=[
            pl.BlockSpec((1, NT, HC), lambda r, t: (r, t, 0)),
            pl.BlockSpec((1, NT, HC), lambda r, t: (r, t, 0)),
        ],
        out_shape=[jax.ShapeDtypeStruct((R, N, HC), f32),
                   jax.ShapeDtypeStruct((R, N, HC), f32)],
    )(h, W_l, b_l.reshape(R, 1, HC), W_r, b_r.reshape(R, 1, HC), W_e,
      rel_emb.reshape(R, 1, REL))


def _ee_body(ea_ref, et_ref, we_ref, ee_ref):
    ea = ea_ref[0]
    etc = et_ref[0]
    acc = jnp.zeros((ET, HC), f32)
    for r in range(R):
        m = (etc == r).astype(f32)
        acc = acc + m * (ea @ we_ref[r, :EA, :])
    ee_ref[0] = acc


def _make_ee(ea3, etc3, W_e):
    return pl.pallas_call(
        _ee_body,
        grid=(TE,),
        in_specs=[
            pl.BlockSpec((1, ET, EA), lambda i: (i, 0, 0)),
            pl.BlockSpec((1, ET, 1), lambda i: (i, 0, 0)),
            pl.BlockSpec((R, ED, HC), lambda i: (0, 0, 0)),
        ],
        out_specs=pl.BlockSpec((1, ET, HC), lambda i: (i, 0, 0)),
        out_shape=jax.ShapeDtypeStruct((TE, ET, HC), f32),
    )(ea3, etc3, W_e)


def _idx_body(src_ref, dst_ref, et_ref, il_ref, ir_ref):
    base = et_ref[0] * N
    il_ref[0] = base + src_ref[0]
    ir_ref[0] = base + dst_ref[0]


def _make_idx(src3, dst3, etc3):
    return pl.pallas_call(
        _idx_body,
        grid=(TE,),
        in_specs=[pl.BlockSpec((1, ET, 1), lambda i: (i, 0, 0))] * 3,
        out_specs=[pl.BlockSpec((1, ET, 1), lambda i: (i, 0, 0))] * 2,
        out_shape=[jax.ShapeDtypeStruct((TE, ET, 1), i32)] * 2,
    )(src3, dst3, etc3)


def _sc_gather2(xlf, xrf, idxl, idxr):
    @functools.partial(
        pl.kernel,
        out_type=[jax.ShapeDtypeStruct((E, HC), f32),
                  jax.ShapeDtypeStruct((E, HC), f32)],
        mesh=_mesh,
        scratch_types=(
            [pltpu.VMEM((CH,), i32)] * 4 + [pltpu.VMEM((CH, HC), f32)] * 4
            + [pltpu.SemaphoreType.DMA] * 8
        ),
    )
    def k(xlf_h, xrf_h, il_h, ir_h, xj_h, xi_h,
          ia0, ia1, ib0, ib1, ra0, ra1, rb0, rb1,
          sa0, sa1, sb0, sb1, wa0, wa1, wb0, wb1):
        ias, ibs = [ia0, ia1], [ib0, ib1]
        ras, rbs = [ra0, ra1], [rb0, rb1]
        sas, sbs = [sa0, sa1], [sb0, sb1]
        was, wbs = [wa0, wa1], [wb0, wb1]
        wid = lax.axis_index("s") * NC + lax.axis_index("c")
        base = wid * EW
        pltpu.sync_copy(il_h.at[pl.ds(base, CH)], ia0)
        pltpu.sync_copy(ir_h.at[pl.ds(base, CH)], ib0)
        pltpu.sync_copy(il_h.at[pl.ds(base + CH, CH)], ia1)
        pltpu.sync_copy(ir_h.at[pl.ds(base + CH, CH)], ib1)
        pltpu.async_copy(xlf_h.at[ia0], ra0, sa0)
        pltpu.async_copy(xrf_h.at[ib0], rb0, sb0)

        def outer(i0, carry):
            for b in range(2):
                i = i0 * 2 + b
                off = base + i * CH
                pltpu.make_async_copy(xlf_h.at[ias[b]], ras[b], sas[b]).wait()
                pltpu.make_async_copy(xrf_h.at[ibs[b]], rbs[b], sbs[b]).wait()
                pltpu.async_copy(ras[b], xj_h.at[pl.ds(off, CH)], was[b])
                pltpu.async_copy(rbs[b], xi_h.at[pl.ds(off, CH)], wbs[b])

                @pl.when(i < NIT - 1)
                def _():
                    @pl.when(i >= 1)
                    def _():
                        pltpu.make_async_copy(
                            ras[1 - b], xj_h.at[pl.ds(off, CH)], was[1 - b]).wait()
                        pltpu.make_async_copy(
                            rbs[1 - b], xi_h.at[pl.ds(off, CH)], wbs[1 - b]).wait()
                    pltpu.async_copy(xlf_h.at[ias[1 - b]], ras[1 - b], sas[1 - b])
                    pltpu.async_copy(xrf_h.at[ibs[1 - b]], rbs[1 - b], sbs[1 - b])

                @pl.when(i < NIT - 2)
                def _():
                    pltpu.sync_copy(il_h.at[pl.ds(off + 2 * CH, CH)], ias[b])
                    pltpu.sync_copy(ir_h.at[pl.ds(off + 2 * CH, CH)], ibs[b])
            return carry

        lax.fori_loop(0, NIT // 2, outer, 0)
        for b in range(2):
            pltpu.make_async_copy(ras[b], xj_h.at[pl.ds(base, CH)], was[b]).wait()
            pltpu.make_async_copy(rbs[b], xi_h.at[pl.ds(base, CH)], wbs[b]).wait()

    return k(xlf, xrf, idxl, idxr)


def _sc_gather1(table, idx, width):
    @functools.partial(
        pl.kernel,
        out_type=jax.ShapeDtypeStruct((E, width), f32),
        mesh=_mesh,
        scratch_types=(
            [pltpu.VMEM((CH,), i32)] * 2 + [pltpu.VMEM((CH, width), f32)] * 2
            + [pltpu.SemaphoreType.DMA] * 4
        ),
    )
    def k(t_h, i_h, o_h, ix0, ix1, r0, r1, sg0, sg1, sw0, sw1):
        ixs, rs = [ix0, ix1], [r0, r1]
        sgs, sws = [sg0, sg1], [sw0, sw1]
        wid = lax.axis_index("s") * NC + lax.axis_index("c")
        base = wid * EW
        pltpu.sync_copy(i_h.at[pl.ds(base, CH)], ix0)
        pltpu.sync_copy(i_h.at[pl.ds(base + CH, CH)], ix1)
        pltpu.async_copy(t_h.at[ix0], r0, sg0)

        def outer(i0, carry):
            for b in range(2):
                i = i0 * 2 + b
                off = base + i * CH
                pltpu.make_async_copy(t_h.at[ixs[b]], rs[b], sgs[b]).wait()
                pltpu.async_copy(rs[b], o_h.at[pl.ds(off, CH)], sws[b])

                @pl.when(i < NIT - 1)
                def _():
                    @pl.when(i >= 1)
                    def _():
                        pltpu.make_async_copy(
                            rs[1 - b], o_h.at[pl.ds(off, CH)], sws[1 - b]).wait()
                    pltpu.async_copy(t_h.at[ixs[1 - b]], rs[1 - b], sgs[1 - b])

                @pl.when(i < NIT - 2)
                def _():
                    pltpu.sync_copy(i_h.at[pl.ds(off + 2 * CH, CH)], ixs[b])
            return carry

        lax.fori_loop(0, NIT // 2, outer, 0)
        for b in range(2):
            pltpu.make_async_copy(rs[b], o_h.at[pl.ds(base, CH)], sws[b]).wait()

    return k(table, idx)


def _sc_scatter_add(vals, dst, zeros, width):
    @functools.partial(
        pl.kernel,
        out_type=jax.ShapeDtypeStruct((NC, NP, width), f32),
        mesh=_mesh,
        scratch_types=(
            [pltpu.VMEM((CH,), i32)] * 2 + [pltpu.VMEM((CH, width), f32)] * 2
            + [pltpu.SemaphoreType.DMA] * 4
            + [pltpu.VMEM_SHARED((NP, width), f32)]
        ),
    )
    def k(vals_h, dst_h, z_h, out_h, ix0, ix1, v0, v1,
          sl0, sl1, sc0, sc1, acc_sh):
        ixs, vs = [ix0, ix1], [v0, v1]
        sls, scs = [sl0, sl1], [sc0, sc1]
        cid = lax.axis_index("c")
        sid = lax.axis_index("s")
        wid = sid * NC + cid
        base = wid * EW
        pltpu.sync_copy(z_h.at[pl.ds(sid * NPS, NPS)],
                        acc_sh.at[pl.ds(sid * NPS, NPS)])
        plsc.subcore_barrier()
        pltpu.sync_copy(dst_h.at[pl.ds(base, CH)], ix0)
        pltpu.async_copy(vals_h.at[pl.ds(base, CH)], v0, sl0)

        def outer(i0, carry):
            for b in range(2):
                i = i0 * 2 + b
                off = base + i * CH
                pltpu.make_async_copy(
                    vals_h.at[pl.ds(off, CH)], vs[b], sls[b]).wait()

                @pl.when(i >= 1)
                def _():
                    pltpu.make_async_copy(
                        vs[1 - b], acc_sh.at[ixs[1 - b]], scs[1 - b]).wait()

                pltpu.async_copy(vs[b], acc_sh.at[ixs[b]], scs[b], add=True)

                @pl.when(i < NIT - 1)
                def _():
                    pltpu.sync_copy(dst_h.at[pl.ds(off + CH, CH)], ixs[1 - b])
                    pltpu.async_copy(
                        vals_h.at[pl.ds(off + CH, CH)], vs[1 - b], sls[1 - b])
            return carry

        lax.fori_loop(0, NIT // 2, outer, 0)
        pltpu.make_async_copy(vs[1], acc_sh.at[ixs[1]], scs[1]).wait()
        plsc.subcore_barrier()
        pltpu.sync_copy(acc_sh.at[pl.ds(sid * NPS, NPS)],
                        out_h.at[cid, pl.ds(sid * NPS, NPS)])

    return k(vals, dst, zeros)


def _logits_body(xj_ref, xi_ref, ee_ref, et_ref, att_ref, p_ref):
    s = xj_ref[0] + xi_ref[0] + ee_ref[0]
    s = jnp.maximum(s, 0.2 * s)
    etc = et_ref[0]
    rr = lax.broadcasted_iota(i32, (ET, R), 1)
    onehot = (etc == rr).astype(f32)
    attsel = onehot @ att_ref[...]
    prod = s * attsel
    logits = prod.reshape(ET, H, C).sum(-1)
    p = jnp.exp(logits)
    p16 = jnp.repeat(onehot, H, axis=1) * jnp.tile(p, (1, R))
    p_ref[0] = jnp.concatenate([p16, jnp.zeros((ET, C - R * H), f32)], axis=1)


def _make_logits(xj3, xi3, ee3, etc3, attf):
    return pl.pallas_call(
        _logits_body,
        grid=(TE,),
        in_specs=[
            pl.BlockSpec((1, ET, HC), lambda i: (i, 0, 0)),
            pl.BlockSpec((1, ET, HC), lambda i: (i, 0, 0)),
            pl.BlockSpec((1, ET, HC), lambda i: (i, 0, 0)),
            pl.BlockSpec((1, ET, 1), lambda i: (i, 0, 0)),
            pl.BlockSpec((R, HC), lambda i: (0, 0)),
        ],
        out_specs=pl.BlockSpec((1, ET, C), lambda i: (i, 0, 0)),
        out_shape=jax.ShapeDtypeStruct((TE, ET, C), f32),
    )(xj3, xi3, ee3, etc3, attf)


def _deninv_body(denp_ref, gate_ref, di_ref):
    den = (denp_ref[0] + denp_ref[1])[:, :R * H]
    g = jax.nn.softmax(gate_ref[...], axis=-1)
    gf = jnp.repeat(g, H, axis=1) / H
    di = gf / (den + 1e-16)
    di_ref[...] = jnp.concatenate([di, jnp.zeros((NPS, C - R * H), f32)], axis=1)


def _make_deninv(denp, gate2):
    return pl.pallas_call(
        _deninv_body,
        grid=(NS,),
        in_specs=[
            pl.BlockSpec((NC, NPS, C), lambda t: (0, t, 0)),
            pl.BlockSpec((1, R), lambda t: (0, 0)),
        ],
        out_specs=pl.BlockSpec((NPS, C), lambda t: (t, 0)),
        out_shape=jax.ShapeDtypeStruct((NP, C), f32),
    )(denp, gate2)


def _msg_body(p_ref, di_ref, xj_ref, v_ref):
    w16 = p_ref[0][:, :R * H] * di_ref[0][:, :R * H]
    ii = lax.broadcasted_iota(i32, (R * H, H), 0)
    jj = lax.broadcasted_iota(i32, (R * H, H), 1)
    sel = (ii % H == jj).astype(f32)
    w4 = w16 @ sel
    xj = xj_ref[0]
    acc = w4[:, 0:1] * xj[:, 0:C]
    for hh in range(1, H):
        acc = acc + w4[:, hh:hh + 1] * xj[:, hh * C:(hh + 1) * C]
    v_ref[0] = acc


def _make_msgs(p3, di3, xj3):
    return pl.pallas_call(
        _msg_body,
        grid=(TE,),
        in_specs=[
            pl.BlockSpec((1, ET, C), lambda i: (i, 0, 0)),
            pl.BlockSpec((1, ET, C), lambda i: (i, 0, 0)),
            pl.BlockSpec((1, ET, HC), lambda i: (i, 0, 0)),
        ],
        out_specs=pl.BlockSpec((1, ET, C), lambda i: (i, 0, 0)),
        out_shape=jax.ShapeDtypeStruct((TE, ET, C), f32),
    )(p3, di3, xj3)


def _final_body(h_ref, op_ref, gate_ref, cb_ref, l1w_ref, l1b_ref,
                l2w_ref, l2b_ref, w1_ref, b1_ref, w2_ref, b2_ref, o_ref):
    g = jax.nn.softmax(gate_ref[...], axis=-1)
    const = g @ cb_ref[...]
    x = h_ref[...] + op_ref[0] + op_ref[1] + const
    mu = jnp.mean(x, axis=-1, keepdims=True)
    var = jnp.mean((x - mu) ** 2, axis=-1, keepdims=True)
    h1 = (x - mu) / jnp.sqrt(var + 1e-5) * l1w_ref[...] + l1b_ref[...]
    t = h1 @ w1_ref[...] + b1_ref[...]
    t = t * jax.nn.sigmoid(t)
    y = t @ w2_ref[...] + b2_ref[...]
    x2 = h1 + y
    mu2 = jnp.mean(x2, axis=-1, keepdims=True)
    var2 = jnp.mean((x2 - mu2) ** 2, axis=-1, keepdims=True)
    o_ref[...] = (x2 - mu2) / jnp.sqrt(var2 + 1e-5) * l2w_ref[...] + l2b_ref[...]


def _make_final(h, outp, gate2, conv_bias, ln1w2, ln1b2, ln2w2, ln2b2,
                ffn_w1, ffn_b12, ffn_w2, ffn_b22):
    return pl.pallas_call(
        _final_body,
        grid=(TN,),
        in_specs=[
            pl.BlockSpec((NT, D), lambda t: (t, 0)),
            pl.BlockSpec((NC, NT, C), lambda t: (0, t, 0)),
            pl.BlockSpec((1, R), lambda t: (0, 0)),
            pl.BlockSpec((R, C), lambda t: (0, 0)),
            pl.BlockSpec((1, D), lambda t: (0, 0)),
            pl.BlockSpec((1, D), lambda t: (0, 0)),
            pl.BlockSpec((1, D), lambda t: (0, 0)),
            pl.BlockSpec((1, D), lambda t: (0, 0)),
            pl.BlockSpec((D, FFN), lambda t: (0, 0)),
            pl.BlockSpec((1, FFN), lambda t: (0, 0)),
            pl.BlockSpec((FFN, D), lambda t: (0, 0)),
            pl.BlockSpec((1, D), lambda t: (0, 0)),
        ],
        out_specs=pl.BlockSpec((NT, D), lambda t: (t, 0)),
        out_shape=jax.ShapeDtypeStruct((N, D), f32),
    )(h, outp, gate2, conv_bias, ln1w2, ln1b2, ln2w2, ln2b2,
      ffn_w1, ffn_b12, ffn_w2, ffn_b22)


def kernel(h, edge_index, edge_attr, edge_type, rel_emb, rel_gate, W_l, b_l,
           W_r, b_r, W_e, att, conv_bias, ln1_w, ln1_b, ln2_w, ln2_b,
           ffn_w1, ffn_b1, ffn_w2, ffn_b2):
    src = edge_index[0]
    dst = edge_index[1]
    etc3 = edge_type.reshape(TE, ET, 1)
    src3 = src.reshape(TE, ET, 1)
    dst3 = dst.reshape(TE, ET, 1)
    ea3 = edge_attr.reshape(TE, ET, EA)
    gate2 = rel_gate.reshape(1, R)

    XL, XR = _make_tables(h, W_l, b_l, W_r, b_r, W_e, rel_emb)
    EE3 = _make_ee(ea3, etc3, W_e)
    idxl3, idxr3 = _make_idx(src3, dst3, etc3)

    XJ, XI = _sc_gather2(XL.reshape(R * N, HC), XR.reshape(R * N, HC),
                         idxl3.reshape(E), idxr3.reshape(E))

    P3 = _make_logits(XJ.reshape(TE, ET, HC), XI.reshape(TE, ET, HC), EE3,
                      etc3, att.reshape(R, HC))

    DENP = _sc_scatter_add(P3.reshape(E, C), dst,
                           jnp.zeros((NP, C), f32), C)
    DENINV = _make_deninv(DENP, gate2)
    DI = _sc_gather1(DENINV, dst, C)

    V3 = _make_msgs(P3, DI.reshape(TE, ET, C), XJ.reshape(TE, ET, HC))
    OUTP = _sc_scatter_add(V3.reshape(E, C), dst, jnp.zeros((NP, C), f32), C)
    OUTP = OUTP[:, :N, :]

    return _make_final(h, OUTP, gate2, conv_bias,
                       ln1_w.reshape(1, D), ln1_b.reshape(1, D),
                       ln2_w.reshape(1, D), ln2_b.reshape(1, D),
                       ffn_w1, ffn_b1.reshape(1, FFN),
                       ffn_w2, ffn_b2.reshape(1, D))


# speedup vs baseline: 1.2586x; 1.2586x over previous
import functools

import jax
import jax.numpy as jnp
from jax import lax
from jax.experimental import pallas as pl
from jax.experimental.pallas import tpu as pltpu
from jax.experimental.pallas import tpu_sc as plsc

N = 10000
E = 320000
D = 128
EA = 16
R = 4
H = 4
C = 128
REL = 8
ED = EA + REL
HC = H * C
FFN = 256

TE, ET = 160, 2000
TN, NT = 10, 1000
NC, NS = 2, 16
NW = NC * NS
EW = E // NW
CH = 40
NIT = EW // CH
NP = 10240
NPS = NP // NS

f32 = jnp.float32
i32 = jnp.int32

_mesh = plsc.VectorSubcoreMesh(core_axis_name="c", subcore_axis_name="s")


def _tables_body(h_ref, wl_ref, bl_ref, wr_ref, br_ref, we_ref, re_ref,
                 xl_ref, xr_ref):
    h = h_ref[...]
    xl_ref[0] = h @ wl_ref[0] + bl_ref[0]
    crel = re_ref[0] @ we_ref[0, EA:, :]
    xr_ref[0] = h @ wr_ref[0] + br_ref[0] + crel


def _make_tables(h, W_l, b_l, W_r, b_r, W_e, rel_emb):
    return pl.pallas_call(
        _tables_body,
        grid=(R, TN),
        in_specs=[
            pl.BlockSpec((NT, D), lambda r, t: (t, 0)),
            pl.BlockSpec((1, D, HC), lambda r, t: (r, 0, 0)),
            pl.BlockSpec((1, 1, HC), lambda r, t: (r, 0, 0)),
            pl.BlockSpec((1, D, HC), lambda r, t: (r, 0, 0)),
            pl.BlockSpec((1, 1, HC), lambda r, t: (r, 0, 0)),
            pl.BlockSpec((1, ED, HC), lambda r, t: (r, 0, 0)),
            pl.BlockSpec((1, 1, REL), lambda r, t: (r, 0, 0)),
        ],
        out_specs=[
            pl.BlockSpec((1, NT, HC), lambda r, t: (r, t, 0)),
            pl.BlockSpec((1, NT, HC), lambda r, t: (r, t, 0)),
        ],
        out_shape=[jax.ShapeDtypeStruct((R, N, HC), f32),
                   jax.ShapeDtypeStruct((R, N, HC), f32)],
    )(h, W_l, b_l.reshape(R, 1, HC), W_r, b_r.reshape(R, 1, HC), W_e,
      rel_emb.reshape(R, 1, REL))


def _onehot(et_ref):
    etrow = et_ref[0]
    ohT = (jnp.broadcast_to(etrow, (R, ET))
           == lax.broadcasted_iota(i32, (R, ET), 0))
    return jnp.transpose(ohT).astype(f32)


def _ee_body(ea_ref, et_ref, we_ref, ee_ref):
    ea = ea_ref[0]
    oh = _onehot(et_ref)
    blk = jnp.concatenate([ea * oh[:, r:r + 1] for r in range(R)], axis=1)
    wcat = jnp.concatenate([we_ref[r, :EA, :] for r in range(R)], axis=0)
    ee_ref[0] = blk @ wcat


def _make_ee(ea3, et3, W_e):
    return pl.pallas_call(
        _ee_body,
        grid=(TE,),
        in_specs=[
            pl.BlockSpec((1, ET, EA), lambda i: (i, 0, 0)),
            pl.BlockSpec((1, 1, ET), lambda i: (i, 0, 0)),
            pl.BlockSpec((R, ED, HC), lambda i: (0, 0, 0)),
        ],
        out_specs=pl.BlockSpec((1, ET, HC), lambda i: (i, 0, 0)),
        out_shape=jax.ShapeDtypeStruct((TE, ET, HC), f32),
    )(ea3, et3, W_e)


def _idx_body(src_ref, dst_ref, et_ref, il_ref, ir_ref):
    base = et_ref[0] * N
    il_ref[0] = base + src_ref[0]
    ir_ref[0] = base + dst_ref[0]


def _make_idx(src3, dst3, et3):
    return pl.pallas_call(
        _idx_body,
        grid=(TE,),
        in_specs=[pl.BlockSpec((1, 1, ET), lambda i: (i, 0, 0))] * 3,
        out_specs=[pl.BlockSpec((1, 1, ET), lambda i: (i, 0, 0))] * 2,
        out_shape=[jax.ShapeDtypeStruct((TE, 1, ET), i32)] * 2,
    )(src3, dst3, et3)


def _sc_gather2(xlf, xrf, idxl, idxr):
    @functools.partial(
        pl.kernel,
        out_type=[jax.ShapeDtypeStruct((E, HC), f32),
                  jax.ShapeDtypeStruct((E, HC), f32)],
        mesh=_mesh,
        scratch_types=(
            [pltpu.VMEM((CH,), i32)] * 4 + [pltpu.VMEM((CH, HC), f32)] * 4
            + [pltpu.SemaphoreType.DMA] * 8
        ),
    )
    def k(xlf_h, xrf_h, il_h, ir_h, xj_h, xi_h,
          ia0, ia1, ib0, ib1, ra0, ra1, rb0, rb1,
          sa0, sa1, sb0, sb1, wa0, wa1, wb0, wb1):
        ias, ibs = [ia0, ia1], [ib0, ib1]
        ras, rbs = [ra0, ra1], [rb0, rb1]
        sas, sbs = [sa0, sa1], [sb0, sb1]
        was, wbs = [wa0, wa1], [wb0, wb1]
        wid = lax.axis_index("s") * NC + lax.axis_index("c")
        base = wid * EW
        pltpu.sync_copy(il_h.at[pl.ds(base, CH)], ia0)
        pltpu.sync_copy(ir_h.at[pl.ds(base, CH)], ib0)
        pltpu.sync_copy(il_h.at[pl.ds(base + CH, CH)], ia1)
        pltpu.sync_copy(ir_h.at[pl.ds(base + CH, CH)], ib1)
        pltpu.async_copy(xlf_h.at[ia0], ra0, sa0)
        pltpu.async_copy(xrf_h.at[ib0], rb0, sb0)

        def outer(i0, carry):
            for b in range(2):
                i = i0 * 2 + b
                off = base + i * CH
                pltpu.make_async_copy(xlf_h.at[ias[b]], ras[b], sas[b]).wait()
                pltpu.make_async_copy(xrf_h.at[ibs[b]], rbs[b], sbs[b]).wait()
                pltpu.async_copy(ras[b], xj_h.at[pl.ds(off, CH)], was[b])
                pltpu.async_copy(rbs[b], xi_h.at[pl.ds(off, CH)], wbs[b])

                @pl.when(i < NIT - 1)
                def _():
                    @pl.when(i >= 1)
                    def _():
                        pltpu.make_async_copy(
                            ras[1 - b], xj_h.at[pl.ds(off, CH)], was[1 - b]).wait()
                        pltpu.make_async_copy(
                            rbs[1 - b], xi_h.at[pl.ds(off, CH)], wbs[1 - b]).wait()
                    pltpu.async_copy(xlf_h.at[ias[1 - b]], ras[1 - b], sas[1 - b])
                    pltpu.async_copy(xrf_h.at[ibs[1 - b]], rbs[1 - b], sbs[1 - b])

                @pl.when(i < NIT - 2)
                def _():
                    pltpu.sync_copy(il_h.at[pl.ds(off + 2 * CH, CH)], ias[b])
                    pltpu.sync_copy(ir_h.at[pl.ds(off + 2 * CH, CH)], ibs[b])
            return carry

        lax.fori_loop(0, NIT // 2, outer, 0)
        for b in range(2):
            pltpu.make_async_copy(ras[b], xj_h.at[pl.ds(base, CH)], was[b]).wait()
            pltpu.make_async_copy(rbs[b], xi_h.at[pl.ds(base, CH)], wbs[b]).wait()

    return k(xlf, xrf, idxl, idxr)


def _sc_gather1(table, idx, width):
    @functools.partial(
        pl.kernel,
        out_type=jax.ShapeDtypeStruct((E, width), f32),
        mesh=_mesh,
        scratch_types=(
            [pltpu.VMEM((CH,), i32)] * 2 + [pltpu.VMEM((CH, width), f32)] * 2
            + [pltpu.SemaphoreType.DMA] * 4
        ),
    )
    def k(t_h, i_h, o_h, ix0, ix1, r0, r1, sg0, sg1, sw0, sw1):
        ixs, rs = [ix0, ix1], [r0, r1]
        sgs, sws = [sg0, sg1], [sw0, sw1]
        wid = lax.axis_index("s") * NC + lax.axis_index("c")
        base = wid * EW
        pltpu.sync_copy(i_h.at[pl.ds(base, CH)], ix0)
        pltpu.sync_copy(i_h.at[pl.ds(base + CH, CH)], ix1)
        pltpu.async_copy(t_h.at[ix0], r0, sg0)

        def outer(i0, carry):
            for b in range(2):
                i = i0 * 2 + b
                off = base + i * CH
                pltpu.make_async_copy(t_h.at[ixs[b]], rs[b], sgs[b]).wait()
                pltpu.async_copy(rs[b], o_h.at[pl.ds(off, CH)], sws[b])

                @pl.when(i < NIT - 1)
                def _():
                    @pl.when(i >= 1)
                    def _():
                        pltpu.make_async_copy(
                            rs[1 - b], o_h.at[pl.ds(off, CH)], sws[1 - b]).wait()
                    pltpu.async_copy(t_h.at[ixs[1 - b]], rs[1 - b], sgs[1 - b])

                @pl.when(i < NIT - 2)
                def _():
                    pltpu.sync_copy(i_h.at[pl.ds(off + 2 * CH, CH)], ixs[b])
            return carry

        lax.fori_loop(0, NIT // 2, outer, 0)
        for b in range(2):
            pltpu.make_async_copy(rs[b], o_h.at[pl.ds(base, CH)], sws[b]).wait()

    return k(table, idx)


def _sc_scatter_add(vals, dst, zeros, width):
    @functools.partial(
        pl.kernel,
        out_type=jax.ShapeDtypeStruct((NC, NP, width), f32),
        mesh=_mesh,
        scratch_types=(
            [pltpu.VMEM((CH,), i32)] * 2 + [pltpu.VMEM((CH, width), f32)] * 2
            + [pltpu.SemaphoreType.DMA] * 4
            + [pltpu.VMEM_SHARED((NP, width), f32)]
        ),
    )
    def k(vals_h, dst_h, z_h, out_h, ix0, ix1, v0, v1,
          sl0, sl1, sc0, sc1, acc_sh):
        ixs, vs = [ix0, ix1], [v0, v1]
        sls, scs = [sl0, sl1], [sc0, sc1]
        cid = lax.axis_index("c")
        sid = lax.axis_index("s")
        wid = sid * NC + cid
        base = wid * EW
        pltpu.sync_copy(z_h.at[pl.ds(sid * NPS, NPS)],
                        acc_sh.at[pl.ds(sid * NPS, NPS)])
        plsc.subcore_barrier()
        pltpu.sync_copy(dst_h.at[pl.ds(base, CH)], ix0)
        pltpu.async_copy(vals_h.at[pl.ds(base, CH)], v0, sl0)

        def outer(i0, carry):
            for b in range(2):
                i = i0 * 2 + b
                off = base + i * CH
                pltpu.make_async_copy(
                    vals_h.at[pl.ds(off, CH)], vs[b], sls[b]).wait()

                @pl.when(i >= 1)
                def _():
                    pltpu.make_async_copy(
                        vs[1 - b], acc_sh.at[ixs[1 - b]], scs[1 - b]).wait()

                pltpu.async_copy(vs[b], acc_sh.at[ixs[b]], scs[b], add=True)

                @pl.when(i < NIT - 1)
                def _():
                    pltpu.sync_copy(dst_h.at[pl.ds(off + CH, CH)], ixs[1 - b])
                    pltpu.async_copy(
                        vals_h.at[pl.ds(off + CH, CH)], vs[1 - b], sls[1 - b])
            return carry

        lax.fori_loop(0, NIT // 2, outer, 0)
        pltpu.make_async_copy(vs[1], acc_sh.at[ixs[1]], scs[1]).wait()
        plsc.subcore_barrier()
        pltpu.sync_copy(acc_sh.at[pl.ds(sid * NPS, NPS)],
                        out_h.at[cid, pl.ds(sid * NPS, NPS)])

    return k(vals, dst, zeros)


def _logits_body(xj_ref, xi_ref, ee_ref, et_ref, att_ref, p_ref):
    s = xj_ref[0] + xi_ref[0] + ee_ref[0]
    s = jnp.maximum(s, 0.2 * s)
    onehot = _onehot(et_ref)
    attsel = onehot @ att_ref[...]
    prod = s * attsel
    logits = prod.reshape(ET, H, C).sum(-1)
    p = jnp.exp(logits)
    p16 = jnp.repeat(onehot, H, axis=1) * jnp.tile(p, (1, R))
    p_ref[0] = jnp.concatenate([p16, jnp.zeros((ET, C - R * H), f32)], axis=1)


def _make_logits(xj3, xi3, ee3, etc3, attf):
    return pl.pallas_call(
        _logits_body,
        grid=(TE,),
        in_specs=[
            pl.BlockSpec((1, ET, HC), lambda i: (i, 0, 0)),
            pl.BlockSpec((1, ET, HC), lambda i: (i, 0, 0)),
            pl.BlockSpec((1, ET, HC), lambda i: (i, 0, 0)),
            pl.BlockSpec((1, 1, ET), lambda i: (i, 0, 0)),
            pl.BlockSpec((R, HC), lambda i: (0, 0)),
        ],
        out_specs=pl.BlockSpec((1, ET, C), lambda i: (i, 0, 0)),
        out_shape=jax.ShapeDtypeStruct((TE, ET, C), f32),
    )(xj3, xi3, ee3, etc3, attf)


def _deninv_body(denp_ref, gate_ref, di_ref):
    den = (denp_ref[0] + denp_ref[1])[:, :R * H]
    g = jax.nn.softmax(gate_ref[...], axis=-1)
    gf = jnp.repeat(g, H, axis=1) / H
    di = gf / (den + 1e-16)
    di_ref[...] = jnp.concatenate([di, jnp.zeros((NPS, C - R * H), f32)], axis=1)


def _make_deninv(denp, gate2):
    return pl.pallas_call(
        _deninv_body,
        grid=(NS,),
        in_specs=[
            pl.BlockSpec((NC, NPS, C), lambda t: (0, t, 0)),
            pl.BlockSpec((1, R), lambda t: (0, 0)),
        ],
        out_specs=pl.BlockSpec((NPS, C), lambda t: (t, 0)),
        out_shape=jax.ShapeDtypeStruct((NP, C), f32),
    )(denp, gate2)


def _msg_body(p_ref, di_ref, xj_ref, v_ref):
    w16 = p_ref[0][:, :R * H] * di_ref[0][:, :R * H]
    ii = lax.broadcasted_iota(i32, (R * H, H), 0)
    jj = lax.broadcasted_iota(i32, (R * H, H), 1)
    sel = (ii % H == jj).astype(f32)
    w4 = w16 @ sel
    xj = xj_ref[0]
    acc = w4[:, 0:1] * xj[:, 0:C]
    for hh in range(1, H):
        acc = acc + w4[:, hh:hh + 1] * xj[:, hh * C:(hh + 1) * C]
    v_ref[0] = acc


def _make_msgs(p3, di3, xj3):
    return pl.pallas_call(
        _msg_body,
        grid=(TE,),
        in_specs=[
            pl.BlockSpec((1, ET, C), lambda i: (i, 0, 0)),
            pl.BlockSpec((1, ET, C), lambda i: (i, 0, 0)),
            pl.BlockSpec((1, ET, HC), lambda i: (i, 0, 0)),
        ],
        out_specs=pl.BlockSpec((1, ET, C), lambda i: (i, 0, 0)),
        out_shape=jax.ShapeDtypeStruct((TE, ET, C), f32),
    )(p3, di3, xj3)


def _final_body(h_ref, op_ref, gate_ref, cb_ref, l1w_ref, l1b_ref,
                l2w_ref, l2b_ref, w1_ref, b1_ref, w2_ref, b2_ref, o_ref):
    g = jax.nn.softmax(gate_ref[...], axis=-1)
    const = g @ cb_ref[...]
    x = h_ref[...] + op_ref[0] + op_ref[1] + const
    mu = jnp.mean(x, axis=-1, keepdims=True)
    var = jnp.mean((x - mu) ** 2, axis=-1, keepdims=True)
    h1 = (x - mu) / jnp.sqrt(var + 1e-5) * l1w_ref[...] + l1b_ref[...]
    t = h1 @ w1_ref[...] + b1_ref[...]
    t = t * jax.nn.sigmoid(t)
    y = t @ w2_ref[...] + b2_ref[...]
    x2 = h1 + y
    mu2 = jnp.mean(x2, axis=-1, keepdims=True)
    var2 = jnp.mean((x2 - mu2) ** 2, axis=-1, keepdims=True)
    o_ref[...] = (x2 - mu2) / jnp.sqrt(var2 + 1e-5) * l2w_ref[...] + l2b_ref[...]


def _make_final(h, outp, gate2, conv_bias, ln1w2, ln1b2, ln2w2, ln2b2,
                ffn_w1, ffn_b12, ffn_w2, ffn_b22):
    return pl.pallas_call(
        _final_body,
        grid=(TN,),
        in_specs=[
            pl.BlockSpec((NT, D), lambda t: (t, 0)),
            pl.BlockSpec((NC, NT, C), lambda t: (0, t, 0)),
            pl.BlockSpec((1, R), lambda t: (0, 0)),
            pl.BlockSpec((R, C), lambda t: (0, 0)),
            pl.BlockSpec((1, D), lambda t: (0, 0)),
            pl.BlockSpec((1, D), lambda t: (0, 0)),
            pl.BlockSpec((1, D), lambda t: (0, 0)),
            pl.BlockSpec((1, D), lambda t: (0, 0)),
            pl.BlockSpec((D, FFN), lambda t: (0, 0)),
            pl.BlockSpec((1, FFN), lambda t: (0, 0)),
            pl.BlockSpec((FFN, D), lambda t: (0, 0)),
            pl.BlockSpec((1, D), lambda t: (0, 0)),
        ],
        out_specs=pl.BlockSpec((NT, D), lambda t: (t, 0)),
        out_shape=jax.ShapeDtypeStruct((N, D), f32),
    )(h, outp, gate2, conv_bias, ln1w2, ln1b2, ln2w2, ln2b2,
      ffn_w1, ffn_b12, ffn_w2, ffn_b22)


def kernel(h, edge_index, edge_attr, edge_type, rel_emb, rel_gate, W_l, b_l,
           W_r, b_r, W_e, att, conv_bias, ln1_w, ln1_b, ln2_w, ln2_b,
           ffn_w1, ffn_b1, ffn_w2, ffn_b2):
    src = edge_index[0]
    dst = edge_index[1]
    et3 = edge_type.reshape(TE, 1, ET)
    src3 = src.reshape(TE, 1, ET)
    dst3 = dst.reshape(TE, 1, ET)
    ea3 = edge_attr.reshape(TE, ET, EA)
    gate2 = rel_gate.reshape(1, R)

    XL, XR = _make_tables(h, W_l, b_l, W_r, b_r, W_e, rel_emb)
    EE3 = _make_ee(ea3, et3, W_e)
    idxl3, idxr3 = _make_idx(src3, dst3, et3)

    XJ, XI = _sc_gather2(XL.reshape(R * N, HC), XR.reshape(R * N, HC),
                         idxl3.reshape(E), idxr3.reshape(E))

    P3 = _make_logits(XJ.reshape(TE, ET, HC), XI.reshape(TE, ET, HC), EE3,
                      et3, att.reshape(R, HC))

    DENP = _sc_scatter_add(P3.reshape(E, C), dst,
                           jnp.zeros((NP, C), f32), C)
    DENINV = _make_deninv(DENP, gate2)
    DI = _sc_gather1(DENINV, dst, C)

    V3 = _make_msgs(P3, DI.reshape(TE, ET, C), XJ.reshape(TE, ET, HC))
    OUTP = _sc_scatter_add(V3.reshape(E, C), dst, jnp.zeros((NP, C), f32), C)
    OUTP = OUTP[:, :N, :]

    return _make_final(h, OUTP, gate2, conv_bias,
                       ln1_w.reshape(1, D), ln1_b.reshape(1, D),
                       ln2_w.reshape(1, D), ln2_b.reshape(1, D),
                       ffn_w1, ffn_b1.reshape(1, FFN),
                       ffn_w2, ffn_b2.reshape(1, D))


# speedup vs baseline: 1.4297x; 1.1359x over previous
import functools

import jax
import jax.numpy as jnp
from jax import lax
from jax.experimental import pallas as pl
from jax.experimental.pallas import tpu as pltpu
from jax.experimental.pallas import tpu_sc as plsc

N = 10000
E = 320000
D = 128
EA = 16
R = 4
H = 4
C = 128
REL = 8
ED = EA + REL
HC = H * C
FFN = 256

TE, ET = 160, 2000
TN, NT = 10, 1000
NC, NS = 2, 16
NW = NC * NS
CH = 40
HALVES = ((0, 192000, 96), (192000, 128000, 64))
NP = 10240
NPS = NP // NS

f32 = jnp.float32
i32 = jnp.int32

_mesh = plsc.VectorSubcoreMesh(core_axis_name="c", subcore_axis_name="s")


def _tables_body(h_ref, wl_ref, bl_ref, wr_ref, br_ref, we_ref, re_ref,
                 xl_ref, xr_ref):
    h = h_ref[...]
    xl_ref[0] = h @ wl_ref[0] + bl_ref[0]
    crel = re_ref[0] @ we_ref[0, EA:, :]
    xr_ref[0] = h @ wr_ref[0] + br_ref[0] + crel


def _make_tables(h, W_l, b_l, W_r, b_r, W_e, rel_emb):
    return pl.pallas_call(
        _tables_body,
        grid=(R, TN),
        in_specs=[
            pl.BlockSpec((NT, D), lambda r, t: (t, 0)),
            pl.BlockSpec((1, D, HC), lambda r, t: (r, 0, 0)),
            pl.BlockSpec((1, 1, HC), lambda r, t: (r, 0, 0)),
            pl.BlockSpec((1, D, HC), lambda r, t: (r, 0, 0)),
            pl.BlockSpec((1, 1, HC), lambda r, t: (r, 0, 0)),
            pl.BlockSpec((1, ED, HC), lambda r, t: (r, 0, 0)),
            pl.BlockSpec((1, 1, REL), lambda r, t: (r, 0, 0)),
        ],
        out_specs=[
            pl.BlockSpec((1, NT, HC), lambda r, t: (r, t, 0)),
            pl.BlockSpec((1, NT, HC), lambda r, t: (r, t, 0)),
        ],
        out_shape=[jax.ShapeDtypeStruct((R, N, HC), f32),
                   jax.ShapeDtypeStruct((R, N, HC), f32)],
    )(h, W_l, b_l.reshape(R, 1, HC), W_r, b_r.reshape(R, 1, HC), W_e,
      rel_emb.reshape(R, 1, REL))


def _onehot(et_ref):
    etrow = et_ref[0]
    ohT = (jnp.broadcast_to(etrow, (R, ET))
           == lax.broadcasted_iota(i32, (R, ET), 0))
    return jnp.transpose(ohT).astype(f32)


def _ee_body(ea_ref, et_ref, we_ref, ee_ref):
    ea = ea_ref[0]
    oh = _onehot(et_ref)
    blk = jnp.concatenate([ea * oh[:, r:r + 1] for r in range(R)], axis=1)
    wcat = jnp.concatenate([we_ref[r, :EA, :] for r in range(R)], axis=0)
    ee_ref[0] = blk @ wcat


def _make_ee(ea3, et3, W_e, te):
    return pl.pallas_call(
        _ee_body,
        grid=(te,),
        in_specs=[
            pl.BlockSpec((1, ET, EA), lambda i: (i, 0, 0)),
            pl.BlockSpec((1, 1, ET), lambda i: (i, 0, 0)),
            pl.BlockSpec((R, ED, HC), lambda i: (0, 0, 0)),
        ],
        out_specs=pl.BlockSpec((1, ET, HC), lambda i: (i, 0, 0)),
        out_shape=jax.ShapeDtypeStruct((te, ET, HC), f32),
    )(ea3, et3, W_e)


def _idx_body(src_ref, dst_ref, et_ref, il_ref, ir_ref):
    base = et_ref[0] * N
    il_ref[0] = base + src_ref[0]
    ir_ref[0] = base + dst_ref[0]


def _make_idx(src3, dst3, et3, te):
    return pl.pallas_call(
        _idx_body,
        grid=(te,),
        in_specs=[pl.BlockSpec((1, 1, ET), lambda i: (i, 0, 0))] * 3,
        out_specs=[pl.BlockSpec((1, 1, ET), lambda i: (i, 0, 0))] * 2,
        out_shape=[jax.ShapeDtypeStruct((te, 1, ET), i32)] * 2,
    )(src3, dst3, et3)


def _sc_gather2(xlf, xrf, idxl, idxr, ne, ew, nit):
    @functools.partial(
        pl.kernel,
        out_type=[jax.ShapeDtypeStruct((ne, HC), f32),
                  jax.ShapeDtypeStruct((ne, HC), f32)],
        mesh=_mesh,
        scratch_types=(
            [pltpu.VMEM((CH,), i32)] * 4 + [pltpu.VMEM((CH, HC), f32)] * 4
            + [pltpu.SemaphoreType.DMA] * 8
        ),
    )
    def k(xlf_h, xrf_h, il_h, ir_h, xj_h, xi_h,
          ia0, ia1, ib0, ib1, ra0, ra1, rb0, rb1,
          sa0, sa1, sb0, sb1, wa0, wa1, wb0, wb1):
        ias, ibs = [ia0, ia1], [ib0, ib1]
        ras, rbs = [ra0, ra1], [rb0, rb1]
        sas, sbs = [sa0, sa1], [sb0, sb1]
        was, wbs = [wa0, wa1], [wb0, wb1]
        wid = lax.axis_index("s") * NC + lax.axis_index("c")
        base = wid * ew
        pltpu.sync_copy(il_h.at[pl.ds(base, CH)], ia0)
        pltpu.sync_copy(ir_h.at[pl.ds(base, CH)], ib0)
        pltpu.sync_copy(il_h.at[pl.ds(base + CH, CH)], ia1)
        pltpu.sync_copy(ir_h.at[pl.ds(base + CH, CH)], ib1)
        pltpu.async_copy(xlf_h.at[ia0], ra0, sa0)
        pltpu.async_copy(xrf_h.at[ib0], rb0, sb0)

        def outer(i0, carry):
            for b in range(2):
                i = i0 * 2 + b
                off = base + i * CH
                pltpu.make_async_copy(xlf_h.at[ias[b]], ras[b], sas[b]).wait()
                pltpu.make_async_copy(xrf_h.at[ibs[b]], rbs[b], sbs[b]).wait()
                pltpu.async_copy(ras[b], xj_h.at[pl.ds(off, CH)], was[b])
                pltpu.async_copy(rbs[b], xi_h.at[pl.ds(off, CH)], wbs[b])

                @pl.when(i < nit - 1)
                def _():
                    @pl.when(i >= 1)
                    def _():
                        pltpu.make_async_copy(
                            ras[1 - b], xj_h.at[pl.ds(off, CH)], was[1 - b]).wait()
                        pltpu.make_async_copy(
                            rbs[1 - b], xi_h.at[pl.ds(off, CH)], wbs[1 - b]).wait()
                    pltpu.async_copy(xlf_h.at[ias[1 - b]], ras[1 - b], sas[1 - b])
                    pltpu.async_copy(xrf_h.at[ibs[1 - b]], rbs[1 - b], sbs[1 - b])

                @pl.when(i < nit - 2)
                def _():
                    pltpu.sync_copy(il_h.at[pl.ds(off + 2 * CH, CH)], ias[b])
                    pltpu.sync_copy(ir_h.at[pl.ds(off + 2 * CH, CH)], ibs[b])
            return carry

        lax.fori_loop(0, nit // 2, outer, 0)
        for b in range(2):
            pltpu.make_async_copy(ras[b], xj_h.at[pl.ds(base, CH)], was[b]).wait()
            pltpu.make_async_copy(rbs[b], xi_h.at[pl.ds(base, CH)], wbs[b]).wait()

    return k(xlf, xrf, idxl, idxr)


def _sc_gather1(table, idx, width, ne, ew, nit):
    @functools.partial(
        pl.kernel,
        out_type=jax.ShapeDtypeStruct((ne, width), f32),
        mesh=_mesh,
        scratch_types=(
            [pltpu.VMEM((CH,), i32)] * 2 + [pltpu.VMEM((CH, width), f32)] * 2
            + [pltpu.SemaphoreType.DMA] * 4
        ),
    )
    def k(t_h, i_h, o_h, ix0, ix1, r0, r1, sg0, sg1, sw0, sw1):
        ixs, rs = [ix0, ix1], [r0, r1]
        sgs, sws = [sg0, sg1], [sw0, sw1]
        wid = lax.axis_index("s") * NC + lax.axis_index("c")
        base = wid * ew
        pltpu.sync_copy(i_h.at[pl.ds(base, CH)], ix0)
        pltpu.sync_copy(i_h.at[pl.ds(base + CH, CH)], ix1)
        pltpu.async_copy(t_h.at[ix0], r0, sg0)

        def outer(i0, carry):
            for b in range(2):
                i = i0 * 2 + b
                off = base + i * CH
                pltpu.make_async_copy(t_h.at[ixs[b]], rs[b], sgs[b]).wait()
                pltpu.async_copy(rs[b], o_h.at[pl.ds(off, CH)], sws[b])

                @pl.when(i < nit - 1)
                def _():
                    @pl.when(i >= 1)
                    def _():
                        pltpu.make_async_copy(
                            rs[1 - b], o_h.at[pl.ds(off, CH)], sws[1 - b]).wait()
                    pltpu.async_copy(t_h.at[ixs[1 - b]], rs[1 - b], sgs[1 - b])

                @pl.when(i < nit - 2)
                def _():
                    pltpu.sync_copy(i_h.at[pl.ds(off + 2 * CH, CH)], ixs[b])
            return carry

        lax.fori_loop(0, nit // 2, outer, 0)
        for b in range(2):
            pltpu.make_async_copy(rs[b], o_h.at[pl.ds(base, CH)], sws[b]).wait()

    return k(table, idx)


def _sc_scatter_add(vals, dst, zeros, width, ne, ew, nit):
    @functools.partial(
        pl.kernel,
        out_type=jax.ShapeDtypeStruct((NC, NP, width), f32),
        mesh=_mesh,
        scratch_types=(
            [pltpu.VMEM((CH,), i32)] * 2 + [pltpu.VMEM((CH, width), f32)] * 2
            + [pltpu.SemaphoreType.DMA] * 4
            + [pltpu.VMEM_SHARED((NP, width), f32)]
        ),
    )
    def k(vals_h, dst_h, z_h, out_h, ix0, ix1, v0, v1,
          sl0, sl1, sc0, sc1, acc_sh):
        ixs, vs = [ix0, ix1], [v0, v1]
        sls, scs = [sl0, sl1], [sc0, sc1]
        cid = lax.axis_index("c")
        sid = lax.axis_index("s")
        wid = sid * NC + cid
        base = wid * ew
        pltpu.sync_copy(z_h.at[pl.ds(sid * NPS, NPS)],
                        acc_sh.at[pl.ds(sid * NPS, NPS)])
        plsc.subcore_barrier()
        pltpu.sync_copy(dst_h.at[pl.ds(base, CH)], ix0)
        pltpu.async_copy(vals_h.at[pl.ds(base, CH)], v0, sl0)

        def outer(i0, carry):
            for b in range(2):
                i = i0 * 2 + b
                off = base + i * CH
                pltpu.make_async_copy(
                    vals_h.at[pl.ds(off, CH)], vs[b], sls[b]).wait()

                @pl.when(i >= 1)
                def _():
                    pltpu.make_async_copy(
                        vs[1 - b], acc_sh.at[ixs[1 - b]], scs[1 - b]).wait()

                pltpu.async_copy(vs[b], acc_sh.at[ixs[b]], scs[b], add=True)

                @pl.when(i < nit - 1)
                def _():
                    pltpu.sync_copy(dst_h.at[pl.ds(off + CH, CH)], ixs[1 - b])
                    pltpu.async_copy(
                        vals_h.at[pl.ds(off + CH, CH)], vs[1 - b], sls[1 - b])
            return carry

        lax.fori_loop(0, nit // 2, outer, 0)
        pltpu.make_async_copy(vs[1], acc_sh.at[ixs[1]], scs[1]).wait()
        plsc.subcore_barrier()
        pltpu.sync_copy(acc_sh.at[pl.ds(sid * NPS, NPS)],
                        out_h.at[cid, pl.ds(sid * NPS, NPS)])

    return k(vals, dst, zeros)


def _logits_body(xj_ref, xi_ref, ee_ref, et_ref, att_ref, p_ref):
    s = xj_ref[0] + xi_ref[0] + ee_ref[0]
    s = jnp.maximum(s, 0.2 * s)
    onehot = _onehot(et_ref)
    attsel = onehot @ att_ref[...]
    prod = s * attsel
    logits = prod.reshape(ET, H, C).sum(-1)
    p = jnp.exp(logits)
    p16 = jnp.repeat(onehot, H, axis=1) * jnp.tile(p, (1, R))
    p_ref[0] = jnp.concatenate([p16, jnp.zeros((ET, C - R * H), f32)], axis=1)


def _make_logits(xj3, xi3, ee3, etc3, attf, te):
    return pl.pallas_call(
        _logits_body,
        grid=(te,),
        in_specs=[
            pl.BlockSpec((1, ET, HC), lambda i: (i, 0, 0)),
            pl.BlockSpec((1, ET, HC), lambda i: (i, 0, 0)),
            pl.BlockSpec((1, ET, HC), lambda i: (i, 0, 0)),
            pl.BlockSpec((1, 1, ET), lambda i: (i, 0, 0)),
            pl.BlockSpec((R, HC), lambda i: (0, 0)),
        ],
        out_specs=pl.BlockSpec((1, ET, C), lambda i: (i, 0, 0)),
        out_shape=jax.ShapeDtypeStruct((te, ET, C), f32),
    )(xj3, xi3, ee3, etc3, attf)


def _deninv_body(denp_ref, denq_ref, gate_ref, di_ref):
    den = (denp_ref[0] + denp_ref[1]
           + denq_ref[0] + denq_ref[1])[:, :R * H]
    g = jax.nn.softmax(gate_ref[...], axis=-1)
    gf = jnp.repeat(g, H, axis=1) / H
    di = gf / (den + 1e-16)
    di_ref[...] = jnp.concatenate([di, jnp.zeros((NPS, C - R * H), f32)], axis=1)


def _make_deninv(denp, denq, gate2):
    return pl.pallas_call(
        _deninv_body,
        grid=(NS,),
        in_specs=[
            pl.BlockSpec((NC, NPS, C), lambda t: (0, t, 0)),
            pl.BlockSpec((NC, NPS, C), lambda t: (0, t, 0)),
            pl.BlockSpec((1, R), lambda t: (0, 0)),
        ],
        out_specs=pl.BlockSpec((NPS, C), lambda t: (t, 0)),
        out_shape=jax.ShapeDtypeStruct((NP, C), f32),
    )(denp, denq, gate2)


def _msg_body(p_ref, di_ref, xj_ref, v_ref):
    w16 = p_ref[0][:, :R * H] * di_ref[0][:, :R * H]
    ii = lax.broadcasted_iota(i32, (R * H, H), 0)
    jj = lax.broadcasted_iota(i32, (R * H, H), 1)
    sel = (ii % H == jj).astype(f32)
    w4 = w16 @ sel
    xj = xj_ref[0]
    acc = w4[:, 0:1] * xj[:, 0:C]
    for hh in range(1, H):
        acc = acc + w4[:, hh:hh + 1] * xj[:, hh * C:(hh + 1) * C]
    v_ref[0] = acc


def _make_msgs(p3, di3, xj3, te):
    return pl.pallas_call(
        _msg_body,
        grid=(te,),
        in_specs=[
            pl.BlockSpec((1, ET, C), lambda i: (i, 0, 0)),
            pl.BlockSpec((1, ET, C), lambda i: (i, 0, 0)),
            pl.BlockSpec((1, ET, HC), lambda i: (i, 0, 0)),
        ],
        out_specs=pl.BlockSpec((1, ET, C), lambda i: (i, 0, 0)),
        out_shape=jax.ShapeDtypeStruct((te, ET, C), f32),
    )(p3, di3, xj3)


def _final_body(h_ref, op_ref, oq_ref, gate_ref, cb_ref, l1w_ref, l1b_ref,
                l2w_ref, l2b_ref, w1_ref, b1_ref, w2_ref, b2_ref, o_ref):
    g = jax.nn.softmax(gate_ref[...], axis=-1)
    const = g @ cb_ref[...]
    x = (h_ref[...] + op_ref[0] + op_ref[1]
         + oq_ref[0] + oq_ref[1] + const)
    mu = jnp.mean(x, axis=-1, keepdims=True)
    var = jnp.mean((x - mu) ** 2, axis=-1, keepdims=True)
    h1 = (x - mu) / jnp.sqrt(var + 1e-5) * l1w_ref[...] + l1b_ref[...]
    t = h1 @ w1_ref[...] + b1_ref[...]
    t = t * jax.nn.sigmoid(t)
    y = t @ w2_ref[...] + b2_ref[...]
    x2 = h1 + y
    mu2 = jnp.mean(x2, axis=-1, keepdims=True)
    var2 = jnp.mean((x2 - mu2) ** 2, axis=-1, keepdims=True)
    o_ref[...] = (x2 - mu2) / jnp.sqrt(var2 + 1e-5) * l2w_ref[...] + l2b_ref[...]


def _make_final(h, outp, outq, gate2, conv_bias, ln1w2, ln1b2, ln2w2, ln2b2,
                ffn_w1, ffn_b12, ffn_w2, ffn_b22):
    return pl.pallas_call(
        _final_body,
        grid=(TN,),
        in_specs=[
            pl.BlockSpec((NT, D), lambda t: (t, 0)),
            pl.BlockSpec((NC, NT, C), lambda t: (0, t, 0)),
            pl.BlockSpec((NC, NT, C), lambda t: (0, t, 0)),
            pl.BlockSpec((1, R), lambda t: (0, 0)),
            pl.BlockSpec((R, C), lambda t: (0, 0)),
            pl.BlockSpec((1, D), lambda t: (0, 0)),
            pl.BlockSpec((1, D), lambda t: (0, 0)),
            pl.BlockSpec((1, D), lambda t: (0, 0)),
            pl.BlockSpec((1, D), lambda t: (0, 0)),
            pl.BlockSpec((D, FFN), lambda t: (0, 0)),
            pl.BlockSpec((1, FFN), lambda t: (0, 0)),
            pl.BlockSpec((FFN, D), lambda t: (0, 0)),
            pl.BlockSpec((1, D), lambda t: (0, 0)),
        ],
        out_specs=pl.BlockSpec((NT, D), lambda t: (t, 0)),
        out_shape=jax.ShapeDtypeStruct((N, D), f32),
    )(h, outp, outq, gate2, conv_bias, ln1w2, ln1b2, ln2w2, ln2b2,
      ffn_w1, ffn_b12, ffn_w2, ffn_b22)


def kernel(h, edge_index, edge_attr, edge_type, rel_emb, rel_gate, W_l, b_l,
           W_r, b_r, W_e, att, conv_bias, ln1_w, ln1_b, ln2_w, ln2_b,
           ffn_w1, ffn_b1, ffn_w2, ffn_b2):
    src = edge_index[0]
    dst = edge_index[1]
    gate2 = rel_gate.reshape(1, R)
    attf = att.reshape(R, HC)

    XL, XR = _make_tables(h, W_l, b_l, W_r, b_r, W_e, rel_emb)
    xlf = XL.reshape(R * N, HC)
    xrf = XR.reshape(R * N, HC)

    et3s, dsts, ee3s, xjs, p3s = [], [], [], [], []
    for (hb, ne, te) in HALVES:
        ew = ne // NW
        et3 = lax.dynamic_slice_in_dim(edge_type, hb, ne).reshape(te, 1, ET)
        src3 = lax.dynamic_slice_in_dim(src, hb, ne).reshape(te, 1, ET)
        dst3 = lax.dynamic_slice_in_dim(dst, hb, ne).reshape(te, 1, ET)
        dsth = lax.dynamic_slice_in_dim(dst, hb, ne)
        ea3 = lax.dynamic_slice_in_dim(edge_attr, hb, ne).reshape(te, ET, EA)
        idxl3, idxr3 = _make_idx(src3, dst3, et3, te)
        XJ, XI = _sc_gather2(xlf, xrf, idxl3.reshape(ne), idxr3.reshape(ne),
                             ne, ew, ew // CH)
        EE3 = _make_ee(ea3, et3, W_e, te)
        P3 = _make_logits(XJ.reshape(te, ET, HC), XI.reshape(te, ET, HC),
                          EE3, et3, attf, te)
        et3s.append(et3)
        dsts.append(dsth)
        xjs.append(XJ)
        p3s.append(P3)

    denps = [
        _sc_scatter_add(p3s[k].reshape(ne, C), dsts[k],
                        jnp.zeros((NP, C), f32), C, ne, ne // NW,
                        ne // NW // CH)
        for k, (hb, ne, te) in enumerate(HALVES)
    ]
    DENINV = _make_deninv(denps[0], denps[1], gate2)

    outps = []
    for k, (hb, ne, te) in enumerate(HALVES):
        ew = ne // NW
        DI = _sc_gather1(DENINV, dsts[k], C, ne, ew, ew // CH)
        V3 = _make_msgs(p3s[k], DI.reshape(te, ET, C),
                        xjs[k].reshape(te, ET, HC), te)
        outps.append(_sc_scatter_add(V3.reshape(ne, C), dsts[k],
                                     jnp.zeros((NP, C), f32), C,
                                     ne, ew, ew // CH))

    return _make_final(h, outps[0][:, :N, :], outps[1][:, :N, :], gate2,
                       conv_bias,
                       ln1_w.reshape(1, D), ln1_b.reshape(1, D),
                       ln2_w.reshape(1, D), ln2_b.reshape(1, D),
                       ffn_w1, ffn_b1.reshape(1, FFN),
                       ffn_w2, ffn_b2.reshape(1, D))


# speedup vs baseline: 1.5413x; 1.0781x over previous
import functools

import jax
import jax.numpy as jnp
from jax import lax
from jax.experimental import pallas as pl
from jax.experimental.pallas import tpu as pltpu
from jax.experimental.pallas import tpu_sc as plsc

N = 10000
E = 320000
D = 128
EA = 16
R = 4
H = 4
C = 128
REL = 8
ED = EA + REL
HC = H * C
FFN = 256

TE, ET = 160, 2000
TN, NT = 10, 1000
NC, NS = 2, 16
NW = NC * NS
CH = 40
HALVES = ((0, 192000, 96), (192000, 128000, 64))
NP = 10240
NPS = NP // NS

f32 = jnp.float32
i32 = jnp.int32

_mesh = plsc.VectorSubcoreMesh(core_axis_name="c", subcore_axis_name="s")


def _tables_body(h_ref, wl_ref, bl_ref, wr_ref, br_ref, we_ref, re_ref,
                 xl_ref, xr_ref):
    h = h_ref[...]
    xl_ref[0] = h @ wl_ref[0] + bl_ref[0]
    crel = re_ref[0] @ we_ref[0, EA:, :]
    xr_ref[0] = h @ wr_ref[0] + br_ref[0] + crel


def _make_tables(h, W_l, b_l, W_r, b_r, W_e, rel_emb):
    return pl.pallas_call(
        _tables_body,
        grid=(R, TN),
        in_specs=[
            pl.BlockSpec((NT, D), lambda r, t: (t, 0)),
            pl.BlockSpec((1, D, HC), lambda r, t: (r, 0, 0)),
            pl.BlockSpec((1, 1, HC), lambda r, t: (r, 0, 0)),
            pl.BlockSpec((1, D, HC), lambda r, t: (r, 0, 0)),
            pl.BlockSpec((1, 1, HC), lambda r, t: (r, 0, 0)),
            pl.BlockSpec((1, ED, HC), lambda r, t: (r, 0, 0)),
            pl.BlockSpec((1, 1, REL), lambda r, t: (r, 0, 0)),
        ],
        out_specs=[
            pl.BlockSpec((1, NT, HC), lambda r, t: (r, t, 0)),
            pl.BlockSpec((1, NT, HC), lambda r, t: (r, t, 0)),
        ],
        out_shape=[jax.ShapeDtypeStruct((R, N, HC), f32),
                   jax.ShapeDtypeStruct((R, N, HC), f32)],
    )(h, W_l, b_l.reshape(R, 1, HC), W_r, b_r.reshape(R, 1, HC), W_e,
      rel_emb.reshape(R, 1, REL))


def _onehot(et_ref):
    etrow = et_ref[0]
    ohT = (jnp.broadcast_to(etrow, (R, ET))
           == lax.broadcasted_iota(i32, (R, ET), 0))
    return jnp.transpose(ohT).astype(f32)


def _ee_body(ea_ref, et_ref, we_ref, ee_ref):
    ea = ea_ref[0]
    oh = _onehot(et_ref)
    blk = jnp.concatenate([ea * oh[:, r:r + 1] for r in range(R)], axis=1)
    wcat = jnp.concatenate([we_ref[r, :EA, :] for r in range(R)], axis=0)
    ee_ref[0] = blk @ wcat


def _make_ee(ea3, et3, W_e, te):
    return pl.pallas_call(
        _ee_body,
        grid=(te,),
        in_specs=[
            pl.BlockSpec((1, ET, EA), lambda i: (i, 0, 0)),
            pl.BlockSpec((1, 1, ET), lambda i: (i, 0, 0)),
            pl.BlockSpec((R, ED, HC), lambda i: (0, 0, 0)),
        ],
        out_specs=pl.BlockSpec((1, ET, HC), lambda i: (i, 0, 0)),
        out_shape=jax.ShapeDtypeStruct((te, ET, HC), f32),
    )(ea3, et3, W_e)


def _idx_body(src_ref, dst_ref, et_ref, il_ref, ir_ref):
    base = et_ref[0] * N
    il_ref[0] = base + src_ref[0]
    ir_ref[0] = base + dst_ref[0]


def _make_idx(src3, dst3, et3, te):
    return pl.pallas_call(
        _idx_body,
        grid=(te,),
        in_specs=[pl.BlockSpec((1, 1, ET), lambda i: (i, 0, 0))] * 3,
        out_specs=[pl.BlockSpec((1, 1, ET), lambda i: (i, 0, 0))] * 2,
        out_shape=[jax.ShapeDtypeStruct((te, 1, ET), i32)] * 2,
    )(src3, dst3, et3)


def _sc_gather2(xlf, xrf, idxl, idxr, ne, ew, nit):
    @functools.partial(
        pl.kernel,
        out_type=[jax.ShapeDtypeStruct((ne, HC), f32),
                  jax.ShapeDtypeStruct((ne, HC), f32)],
        mesh=_mesh,
        scratch_types=(
            [pltpu.VMEM((CH,), i32)] * 4 + [pltpu.VMEM((CH, HC), f32)] * 4
            + [pltpu.SemaphoreType.DMA] * 8
        ),
    )
    def k(xlf_h, xrf_h, il_h, ir_h, xj_h, xi_h,
          ia0, ia1, ib0, ib1, ra0, ra1, rb0, rb1,
          sa0, sa1, sb0, sb1, wa0, wa1, wb0, wb1):
        ias, ibs = [ia0, ia1], [ib0, ib1]
        ras, rbs = [ra0, ra1], [rb0, rb1]
        sas, sbs = [sa0, sa1], [sb0, sb1]
        was, wbs = [wa0, wa1], [wb0, wb1]
        wid = lax.axis_index("s") * NC + lax.axis_index("c")
        base = wid * ew
        pltpu.sync_copy(il_h.at[pl.ds(base, CH)], ia0)
        pltpu.sync_copy(ir_h.at[pl.ds(base, CH)], ib0)
        pltpu.sync_copy(il_h.at[pl.ds(base + CH, CH)], ia1)
        pltpu.sync_copy(ir_h.at[pl.ds(base + CH, CH)], ib1)
        pltpu.async_copy(xlf_h.at[ia0], ra0, sa0)
        pltpu.async_copy(xrf_h.at[ib0], rb0, sb0)

        def outer(i0, carry):
            for b in range(2):
                i = i0 * 2 + b
                off = base + i * CH
                pltpu.make_async_copy(xlf_h.at[ias[b]], ras[b], sas[b]).wait()
                pltpu.make_async_copy(xrf_h.at[ibs[b]], rbs[b], sbs[b]).wait()
                pltpu.async_copy(ras[b], xj_h.at[pl.ds(off, CH)], was[b])
                pltpu.async_copy(rbs[b], xi_h.at[pl.ds(off, CH)], wbs[b])

                @pl.when(i < nit - 1)
                def _():
                    @pl.when(i >= 1)
                    def _():
                        pltpu.make_async_copy(
                            ras[1 - b], xj_h.at[pl.ds(off, CH)], was[1 - b]).wait()
                        pltpu.make_async_copy(
                            rbs[1 - b], xi_h.at[pl.ds(off, CH)], wbs[1 - b]).wait()
                    pltpu.async_copy(xlf_h.at[ias[1 - b]], ras[1 - b], sas[1 - b])
                    pltpu.async_copy(xrf_h.at[ibs[1 - b]], rbs[1 - b], sbs[1 - b])

                @pl.when(i < nit - 2)
                def _():
                    pltpu.sync_copy(il_h.at[pl.ds(off + 2 * CH, CH)], ias[b])
                    pltpu.sync_copy(ir_h.at[pl.ds(off + 2 * CH, CH)], ibs[b])
            return carry

        lax.fori_loop(0, nit // 2, outer, 0)
        for b in range(2):
            pltpu.make_async_copy(ras[b], xj_h.at[pl.ds(base, CH)], was[b]).wait()
            pltpu.make_async_copy(rbs[b], xi_h.at[pl.ds(base, CH)], wbs[b]).wait()

    return k(xlf, xrf, idxl, idxr)


def _sc_gather1(table, idx, width, ne, ew, nit):
    @functools.partial(
        pl.kernel,
        out_type=jax.ShapeDtypeStruct((ne, width), f32),
        mesh=_mesh,
        scratch_types=(
            [pltpu.VMEM((CH,), i32)] * 2 + [pltpu.VMEM((CH, width), f32)] * 2
            + [pltpu.SemaphoreType.DMA] * 4
        ),
    )
    def k(t_h, i_h, o_h, ix0, ix1, r0, r1, sg0, sg1, sw0, sw1):
        ixs, rs = [ix0, ix1], [r0, r1]
        sgs, sws = [sg0, sg1], [sw0, sw1]
        wid = lax.axis_index("s") * NC + lax.axis_index("c")
        base = wid * ew
        pltpu.sync_copy(i_h.at[pl.ds(base, CH)], ix0)
        pltpu.sync_copy(i_h.at[pl.ds(base + CH, CH)], ix1)
        pltpu.async_copy(t_h.at[ix0], r0, sg0)

        def outer(i0, carry):
            for b in range(2):
                i = i0 * 2 + b
                off = base + i * CH
                pltpu.make_async_copy(t_h.at[ixs[b]], rs[b], sgs[b]).wait()
                pltpu.async_copy(rs[b], o_h.at[pl.ds(off, CH)], sws[b])

                @pl.when(i < nit - 1)
                def _():
                    @pl.when(i >= 1)
                    def _():
                        pltpu.make_async_copy(
                            rs[1 - b], o_h.at[pl.ds(off, CH)], sws[1 - b]).wait()
                    pltpu.async_copy(t_h.at[ixs[1 - b]], rs[1 - b], sgs[1 - b])

                @pl.when(i < nit - 2)
                def _():
                    pltpu.sync_copy(i_h.at[pl.ds(off + 2 * CH, CH)], ixs[b])
            return carry

        lax.fori_loop(0, nit // 2, outer, 0)
        for b in range(2):
            pltpu.make_async_copy(rs[b], o_h.at[pl.ds(base, CH)], sws[b]).wait()

    return k(table, idx)


def _sc_scatter_add(vals, dst, zeros, width, ne, ew, nit):
    @functools.partial(
        pl.kernel,
        out_type=jax.ShapeDtypeStruct((NC, NP, width), f32),
        mesh=_mesh,
        scratch_types=(
            [pltpu.VMEM((CH,), i32)] * 2 + [pltpu.VMEM((CH, width), f32)] * 2
            + [pltpu.SemaphoreType.DMA] * 4
            + [pltpu.VMEM_SHARED((NP, width), f32)]
        ),
    )
    def k(vals_h, dst_h, z_h, out_h, ix0, ix1, v0, v1,
          sl0, sl1, sc0, sc1, acc_sh):
        ixs, vs = [ix0, ix1], [v0, v1]
        sls, scs = [sl0, sl1], [sc0, sc1]
        cid = lax.axis_index("c")
        sid = lax.axis_index("s")
        wid = sid * NC + cid
        base = wid * ew
        pltpu.sync_copy(z_h.at[pl.ds(sid * NPS, NPS)],
                        acc_sh.at[pl.ds(sid * NPS, NPS)])
        plsc.subcore_barrier()
        pltpu.sync_copy(dst_h.at[pl.ds(base, CH)], ix0)
        pltpu.async_copy(vals_h.at[pl.ds(base, CH)], v0, sl0)

        def outer(i0, carry):
            for b in range(2):
                i = i0 * 2 + b
                off = base + i * CH
                pltpu.make_async_copy(
                    vals_h.at[pl.ds(off, CH)], vs[b], sls[b]).wait()

                @pl.when(i >= 1)
                def _():
                    pltpu.make_async_copy(
                        vs[1 - b], acc_sh.at[ixs[1 - b]], scs[1 - b]).wait()

                pltpu.async_copy(vs[b], acc_sh.at[ixs[b]], scs[b], add=True)

                @pl.when(i < nit - 1)
                def _():
                    pltpu.sync_copy(dst_h.at[pl.ds(off + CH, CH)], ixs[1 - b])
                    pltpu.async_copy(
                        vals_h.at[pl.ds(off + CH, CH)], vs[1 - b], sls[1 - b])
            return carry

        lax.fori_loop(0, nit // 2, outer, 0)
        pltpu.make_async_copy(vs[1], acc_sh.at[ixs[1]], scs[1]).wait()
        plsc.subcore_barrier()
        pltpu.sync_copy(acc_sh.at[pl.ds(sid * NPS, NPS)],
                        out_h.at[cid, pl.ds(sid * NPS, NPS)])

    return k(vals, dst, zeros)


def _logits_body(xj_ref, xi_ref, ee_ref, et_ref, att_ref, p_ref):
    s = xj_ref[0] + xi_ref[0] + ee_ref[0]
    s = jnp.maximum(s, 0.2 * s)
    onehot = _onehot(et_ref)
    l16 = s @ att_ref[...]
    p16 = jnp.repeat(onehot, H, axis=1) * jnp.exp(l16)
    p_ref[0] = jnp.concatenate([p16, jnp.zeros((ET, C - R * H), f32)], axis=1)


def _make_logits(xj3, xi3, ee3, etc3, attm, te):
    return pl.pallas_call(
        _logits_body,
        grid=(te,),
        in_specs=[
            pl.BlockSpec((1, ET, HC), lambda i: (i, 0, 0)),
            pl.BlockSpec((1, ET, HC), lambda i: (i, 0, 0)),
            pl.BlockSpec((1, ET, HC), lambda i: (i, 0, 0)),
            pl.BlockSpec((1, 1, ET), lambda i: (i, 0, 0)),
            pl.BlockSpec((HC, R * H), lambda i: (0, 0)),
        ],
        out_specs=pl.BlockSpec((1, ET, C), lambda i: (i, 0, 0)),
        out_shape=jax.ShapeDtypeStruct((te, ET, C), f32),
    )(xj3, xi3, ee3, etc3, attm)


def _deninv_body(denp_ref, denq_ref, gate_ref, di_ref):
    den = (denp_ref[0] + denp_ref[1]
           + denq_ref[0] + denq_ref[1])[:, :R * H]
    g = jax.nn.softmax(gate_ref[...], axis=-1)
    gf = jnp.repeat(g, H, axis=1) / H
    di = gf / (den + 1e-16)
    di_ref[...] = jnp.concatenate([di, jnp.zeros((NPS, C - R * H), f32)], axis=1)


def _make_deninv(denp, denq, gate2):
    return pl.pallas_call(
        _deninv_body,
        grid=(NS,),
        in_specs=[
            pl.BlockSpec((NC, NPS, C), lambda t: (0, t, 0)),
            pl.BlockSpec((NC, NPS, C), lambda t: (0, t, 0)),
            pl.BlockSpec((1, R), lambda t: (0, 0)),
        ],
        out_specs=pl.BlockSpec((NPS, C), lambda t: (t, 0)),
        out_shape=jax.ShapeDtypeStruct((NP, C), f32),
    )(denp, denq, gate2)


def _msg_body(p_ref, di_ref, xj_ref, v_ref):
    w16 = p_ref[0][:, :R * H] * di_ref[0][:, :R * H]
    ii = lax.broadcasted_iota(i32, (R * H, H), 0)
    jj = lax.broadcasted_iota(i32, (R * H, H), 1)
    sel = (ii % H == jj).astype(f32)
    w4 = w16 @ sel
    xj = xj_ref[0]
    acc = w4[:, 0:1] * xj[:, 0:C]
    for hh in range(1, H):
        acc = acc + w4[:, hh:hh + 1] * xj[:, hh * C:(hh + 1) * C]
    v_ref[0] = acc


def _make_msgs(p3, di3, xj3, te):
    return pl.pallas_call(
        _msg_body,
        grid=(te,),
        in_specs=[
            pl.BlockSpec((1, ET, C), lambda i: (i, 0, 0)),
            pl.BlockSpec((1, ET, C), lambda i: (i, 0, 0)),
            pl.BlockSpec((1, ET, HC), lambda i: (i, 0, 0)),
        ],
        out_specs=pl.BlockSpec((1, ET, C), lambda i: (i, 0, 0)),
        out_shape=jax.ShapeDtypeStruct((te, ET, C), f32),
    )(p3, di3, xj3)


def _final_body(h_ref, op_ref, oq_ref, gate_ref, cb_ref, l1w_ref, l1b_ref,
                l2w_ref, l2b_ref, w1_ref, b1_ref, w2_ref, b2_ref, o_ref):
    g = jax.nn.softmax(gate_ref[...], axis=-1)
    const = g @ cb_ref[...]
    x = (h_ref[...] + op_ref[0] + op_ref[1]
         + oq_ref[0] + oq_ref[1] + const)
    mu = jnp.mean(x, axis=-1, keepdims=True)
    var = jnp.mean((x - mu) ** 2, axis=-1, keepdims=True)
    h1 = (x - mu) / jnp.sqrt(var + 1e-5) * l1w_ref[...] + l1b_ref[...]
    t = h1 @ w1_ref[...] + b1_ref[...]
    t = t * jax.nn.sigmoid(t)
    y = t @ w2_ref[...] + b2_ref[...]
    x2 = h1 + y
    mu2 = jnp.mean(x2, axis=-1, keepdims=True)
    var2 = jnp.mean((x2 - mu2) ** 2, axis=-1, keepdims=True)
    o_ref[...] = (x2 - mu2) / jnp.sqrt(var2 + 1e-5) * l2w_ref[...] + l2b_ref[...]


def _make_final(h, outp, outq, gate2, conv_bias, ln1w2, ln1b2, ln2w2, ln2b2,
                ffn_w1, ffn_b12, ffn_w2, ffn_b22):
    return pl.pallas_call(
        _final_body,
        grid=(TN,),
        in_specs=[
            pl.BlockSpec((NT, D), lambda t: (t, 0)),
            pl.BlockSpec((NC, NT, C), lambda t: (0, t, 0)),
            pl.BlockSpec((NC, NT, C), lambda t: (0, t, 0)),
            pl.BlockSpec((1, R), lambda t: (0, 0)),
            pl.BlockSpec((R, C), lambda t: (0, 0)),
            pl.BlockSpec((1, D), lambda t: (0, 0)),
            pl.BlockSpec((1, D), lambda t: (0, 0)),
            pl.BlockSpec((1, D), lambda t: (0, 0)),
            pl.BlockSpec((1, D), lambda t: (0, 0)),
            pl.BlockSpec((D, FFN), lambda t: (0, 0)),
            pl.BlockSpec((1, FFN), lambda t: (0, 0)),
            pl.BlockSpec((FFN, D), lambda t: (0, 0)),
            pl.BlockSpec((1, D), lambda t: (0, 0)),
        ],
        out_specs=pl.BlockSpec((NT, D), lambda t: (t, 0)),
        out_shape=jax.ShapeDtypeStruct((N, D), f32),
    )(h, outp, outq, gate2, conv_bias, ln1w2, ln1b2, ln2w2, ln2b2,
      ffn_w1, ffn_b12, ffn_w2, ffn_b22)


def kernel(h, edge_index, edge_attr, edge_type, rel_emb, rel_gate, W_l, b_l,
           W_r, b_r, W_e, att, conv_bias, ln1_w, ln1_b, ln2_w, ln2_b,
           ffn_w1, ffn_b1, ffn_w2, ffn_b2):
    src = edge_index[0]
    dst = edge_index[1]
    gate2 = rel_gate.reshape(1, R)
    attm = jnp.einsum('rhc,hg->hcrg', att, jnp.eye(H, dtype=f32))
    attm = attm.reshape(HC, R * H)

    XL, XR = _make_tables(h, W_l, b_l, W_r, b_r, W_e, rel_emb)
    xlf = XL.reshape(R * N, HC)
    xrf = XR.reshape(R * N, HC)

    et3s, dsts, ee3s, xjs, p3s = [], [], [], [], []
    for (hb, ne, te) in HALVES:
        ew = ne // NW
        et3 = lax.dynamic_slice_in_dim(edge_type, hb, ne).reshape(te, 1, ET)
        src3 = lax.dynamic_slice_in_dim(src, hb, ne).reshape(te, 1, ET)
        dst3 = lax.dynamic_slice_in_dim(dst, hb, ne).reshape(te, 1, ET)
        dsth = lax.dynamic_slice_in_dim(dst, hb, ne)
        ea3 = lax.dynamic_slice_in_dim(edge_attr, hb, ne).reshape(te, ET, EA)
        idxl3, idxr3 = _make_idx(src3, dst3, et3, te)
        XJ, XI = _sc_gather2(xlf, xrf, idxl3.reshape(ne), idxr3.reshape(ne),
                             ne, ew, ew // CH)
        EE3 = _make_ee(ea3, et3, W_e, te)
        P3 = _make_logits(XJ.reshape(te, ET, HC), XI.reshape(te, ET, HC),
                          EE3, et3, attm, te)
        et3s.append(et3)
        dsts.append(dsth)
        xjs.append(XJ)
        p3s.append(P3)

    denps = [
        _sc_scatter_add(p3s[k].reshape(ne, C), dsts[k],
                        jnp.zeros((NP, C), f32), C, ne, ne // NW,
                        ne // NW // CH)
        for k, (hb, ne, te) in enumerate(HALVES)
    ]
    DENINV = _make_deninv(denps[0], denps[1], gate2)

    outps = []
    for k, (hb, ne, te) in enumerate(HALVES):
        ew = ne // NW
        DI = _sc_gather1(DENINV, dsts[k], C, ne, ew, ew // CH)
        V3 = _make_msgs(p3s[k], DI.reshape(te, ET, C),
                        xjs[k].reshape(te, ET, HC), te)
        outps.append(_sc_scatter_add(V3.reshape(ne, C), dsts[k],
                                     jnp.zeros((NP, C), f32), C,
                                     ne, ew, ew // CH))

    return _make_final(h, outps[0][:, :N, :], outps[1][:, :N, :], gate2,
                       conv_bias,
                       ln1_w.reshape(1, D), ln1_b.reshape(1, D),
                       ln2_w.reshape(1, D), ln2_b.reshape(1, D),
                       ffn_w1, ffn_b1.reshape(1, FFN),
                       ffn_w2, ffn_b2.reshape(1, D))


# speedup vs baseline: 1.7520x; 1.1367x over previous
import functools

import jax
import jax.numpy as jnp
from jax import lax
from jax.experimental import pallas as pl
from jax.experimental.pallas import tpu as pltpu
from jax.experimental.pallas import tpu_sc as plsc

N = 10000
E = 320000
D = 128
EA = 16
R = 4
H = 4
C = 128
REL = 8
ED = EA + REL
HC = H * C
FFN = 256

TE, ET = 160, 2000
TN, NT = 10, 1000
NC, NS = 2, 16
NW = NC * NS
CH = 40
HALVES = ((0, 192000, 96), (192000, 128000, 64))


def _chs(ne):
    return 120 if ne == 192000 else 80
NP = 10240
NPS = NP // NS

f32 = jnp.float32
bf16 = jnp.bfloat16
i32 = jnp.int32

_mesh = plsc.VectorSubcoreMesh(core_axis_name="c", subcore_axis_name="s")


def _tables_body(h_ref, wl_ref, bl_ref, wr_ref, br_ref, we_ref, re_ref,
                 xl_ref, xr_ref):
    h = h_ref[...]
    xl_ref[0] = h @ wl_ref[0] + bl_ref[0]
    crel = re_ref[0] @ we_ref[0, EA:, :]
    xr_ref[0] = h @ wr_ref[0] + br_ref[0] + crel


def _make_tables(h, W_l, b_l, W_r, b_r, W_e, rel_emb):
    return pl.pallas_call(
        _tables_body,
        grid=(R, TN),
        in_specs=[
            pl.BlockSpec((NT, D), lambda r, t: (t, 0)),
            pl.BlockSpec((1, D, HC), lambda r, t: (r, 0, 0)),
            pl.BlockSpec((1, 1, HC), lambda r, t: (r, 0, 0)),
            pl.BlockSpec((1, D, HC), lambda r, t: (r, 0, 0)),
            pl.BlockSpec((1, 1, HC), lambda r, t: (r, 0, 0)),
            pl.BlockSpec((1, ED, HC), lambda r, t: (r, 0, 0)),
            pl.BlockSpec((1, 1, REL), lambda r, t: (r, 0, 0)),
        ],
        out_specs=[
            pl.BlockSpec((1, NT, HC), lambda r, t: (r, t, 0)),
            pl.BlockSpec((1, NT, HC), lambda r, t: (r, t, 0)),
        ],
        out_shape=[jax.ShapeDtypeStruct((R, N, HC), f32),
                   jax.ShapeDtypeStruct((R, N, HC), f32)],
    )(h, W_l, b_l.reshape(R, 1, HC), W_r, b_r.reshape(R, 1, HC), W_e,
      rel_emb.reshape(R, 1, REL))


def _onehot(et_ref):
    etrow = et_ref[0]
    ohT = (jnp.broadcast_to(etrow, (R, ET))
           == lax.broadcasted_iota(i32, (R, ET), 0))
    return jnp.transpose(ohT).astype(f32)


def _ee_body(ea_ref, et_ref, we_ref, ee_ref):
    ea = ea_ref[0]
    oh = _onehot(et_ref)
    blk = jnp.concatenate([ea * oh[:, r:r + 1] for r in range(R)], axis=1)
    wcat = jnp.concatenate([we_ref[r, :EA, :] for r in range(R)], axis=0)
    ee_ref[0] = (blk @ wcat).astype(bf16)


def _make_ee(ea3, et3, W_e, te):
    return pl.pallas_call(
        _ee_body,
        grid=(te,),
        in_specs=[
            pl.BlockSpec((1, ET, EA), lambda i: (i, 0, 0)),
            pl.BlockSpec((1, 1, ET), lambda i: (i, 0, 0)),
            pl.BlockSpec((R, ED, HC), lambda i: (0, 0, 0)),
        ],
        out_specs=pl.BlockSpec((1, ET, HC), lambda i: (i, 0, 0)),
        out_shape=jax.ShapeDtypeStruct((te, ET, HC), bf16),
    )(ea3, et3, W_e)


def _idx_body(src_ref, dst_ref, et_ref, il_ref, ir_ref):
    base = et_ref[0] * N
    il_ref[0] = base + src_ref[0]
    ir_ref[0] = base + dst_ref[0]


def _make_idx(src3, dst3, et3, te):
    return pl.pallas_call(
        _idx_body,
        grid=(te,),
        in_specs=[pl.BlockSpec((1, 1, ET), lambda i: (i, 0, 0))] * 3,
        out_specs=[pl.BlockSpec((1, 1, ET), lambda i: (i, 0, 0))] * 2,
        out_shape=[jax.ShapeDtypeStruct((te, 1, ET), i32)] * 2,
    )(src3, dst3, et3)


def _sc_gather2(xlf, xrf, idxl, idxr, ne, ew, nit):
    @functools.partial(
        pl.kernel,
        out_type=[jax.ShapeDtypeStruct((ne, HC), f32),
                  jax.ShapeDtypeStruct((ne, HC), f32)],
        mesh=_mesh,
        scratch_types=(
            [pltpu.VMEM((CH,), i32)] * 4 + [pltpu.VMEM((CH, HC), f32)] * 4
            + [pltpu.SemaphoreType.DMA] * 8
        ),
    )
    def k(xlf_h, xrf_h, il_h, ir_h, xj_h, xi_h,
          ia0, ia1, ib0, ib1, ra0, ra1, rb0, rb1,
          sa0, sa1, sb0, sb1, wa0, wa1, wb0, wb1):
        ias, ibs = [ia0, ia1], [ib0, ib1]
        ras, rbs = [ra0, ra1], [rb0, rb1]
        sas, sbs = [sa0, sa1], [sb0, sb1]
        was, wbs = [wa0, wa1], [wb0, wb1]
        wid = lax.axis_index("s") * NC + lax.axis_index("c")
        base = wid * ew
        pltpu.sync_copy(il_h.at[pl.ds(base, CH)], ia0)
        pltpu.sync_copy(ir_h.at[pl.ds(base, CH)], ib0)
        pltpu.sync_copy(il_h.at[pl.ds(base + CH, CH)], ia1)
        pltpu.sync_copy(ir_h.at[pl.ds(base + CH, CH)], ib1)
        pltpu.async_copy(xlf_h.at[ia0], ra0, sa0)
        pltpu.async_copy(xrf_h.at[ib0], rb0, sb0)

        def outer(i0, carry):
            for b in range(2):
                i = i0 * 2 + b
                off = base + i * CH
                pltpu.make_async_copy(xlf_h.at[ias[b]], ras[b], sas[b]).wait()
                pltpu.make_async_copy(xrf_h.at[ibs[b]], rbs[b], sbs[b]).wait()
                pltpu.async_copy(ras[b], xj_h.at[pl.ds(off, CH)], was[b])
                pltpu.async_copy(rbs[b], xi_h.at[pl.ds(off, CH)], wbs[b])

                @pl.when(i < nit - 1)
                def _():
                    @pl.when(i >= 1)
                    def _():
                        pltpu.make_async_copy(
                            ras[1 - b], xj_h.at[pl.ds(off, CH)], was[1 - b]).wait()
                        pltpu.make_async_copy(
                            rbs[1 - b], xi_h.at[pl.ds(off, CH)], wbs[1 - b]).wait()
                    pltpu.async_copy(xlf_h.at[ias[1 - b]], ras[1 - b], sas[1 - b])
                    pltpu.async_copy(xrf_h.at[ibs[1 - b]], rbs[1 - b], sbs[1 - b])

                @pl.when(i < nit - 2)
                def _():
                    pltpu.sync_copy(il_h.at[pl.ds(off + 2 * CH, CH)], ias[b])
                    pltpu.sync_copy(ir_h.at[pl.ds(off + 2 * CH, CH)], ibs[b])
            return carry

        lax.fori_loop(0, nit // 2, outer, 0)
        for b in range(2):
            pltpu.make_async_copy(ras[b], xj_h.at[pl.ds(base, CH)], was[b]).wait()
            pltpu.make_async_copy(rbs[b], xi_h.at[pl.ds(base, CH)], wbs[b]).wait()

    return k(xlf, xrf, idxl, idxr)


def _sc_gather1(table, idx, width, ne, ew, nit, ch):
    @functools.partial(
        pl.kernel,
        out_type=jax.ShapeDtypeStruct((ne, width), f32),
        mesh=_mesh,
        scratch_types=(
            [pltpu.VMEM((ch,), i32)] * 2 + [pltpu.VMEM((ch, width), f32)] * 2
            + [pltpu.SemaphoreType.DMA] * 4
        ),
    )
    def k(t_h, i_h, o_h, ix0, ix1, r0, r1, sg0, sg1, sw0, sw1):
        ixs, rs = [ix0, ix1], [r0, r1]
        sgs, sws = [sg0, sg1], [sw0, sw1]
        wid = lax.axis_index("s") * NC + lax.axis_index("c")
        base = wid * ew
        pltpu.sync_copy(i_h.at[pl.ds(base, ch)], ix0)
        pltpu.sync_copy(i_h.at[pl.ds(base + ch, ch)], ix1)
        pltpu.async_copy(t_h.at[ix0], r0, sg0)

        def outer(i0, carry):
            for b in range(2):
                i = i0 * 2 + b
                off = base + i * ch
                pltpu.make_async_copy(t_h.at[ixs[b]], rs[b], sgs[b]).wait()
                pltpu.async_copy(rs[b], o_h.at[pl.ds(off, ch)], sws[b])

                @pl.when(i < nit - 1)
                def _():
                    @pl.when(i >= 1)
                    def _():
                        pltpu.make_async_copy(
                            rs[1 - b], o_h.at[pl.ds(off, ch)], sws[1 - b]).wait()
                    pltpu.async_copy(t_h.at[ixs[1 - b]], rs[1 - b], sgs[1 - b])

                @pl.when(i < nit - 2)
                def _():
                    pltpu.sync_copy(i_h.at[pl.ds(off + 2 * ch, ch)], ixs[b])
            return carry

        lax.fori_loop(0, nit // 2, outer, 0)
        for b in range(2):
            pltpu.make_async_copy(rs[b], o_h.at[pl.ds(base, ch)], sws[b]).wait()

    return k(table, idx)


def _sc_scatter_add(vals, dst, zeros, width, ne, ew, nit, ch):
    @functools.partial(
        pl.kernel,
        out_type=jax.ShapeDtypeStruct((NC, NP, width), f32),
        mesh=_mesh,
        scratch_types=(
            [pltpu.VMEM((ch,), i32)] * 2 + [pltpu.VMEM((ch, width), f32)] * 2
            + [pltpu.SemaphoreType.DMA] * 4
            + [pltpu.VMEM_SHARED((NP, width), f32)]
        ),
    )
    def k(vals_h, dst_h, z_h, out_h, ix0, ix1, v0, v1,
          sl0, sl1, sc0, sc1, acc_sh):
        ixs, vs = [ix0, ix1], [v0, v1]
        sls, scs = [sl0, sl1], [sc0, sc1]
        cid = lax.axis_index("c")
        sid = lax.axis_index("s")
        wid = sid * NC + cid
        base = wid * ew
        pltpu.sync_copy(z_h.at[pl.ds(sid * NPS, NPS)],
                        acc_sh.at[pl.ds(sid * NPS, NPS)])
        plsc.subcore_barrier()
        pltpu.sync_copy(dst_h.at[pl.ds(base, ch)], ix0)
        pltpu.async_copy(vals_h.at[pl.ds(base, ch)], v0, sl0)

        def outer(i0, carry):
            for b in range(2):
                i = i0 * 2 + b
                off = base + i * ch
                pltpu.make_async_copy(
                    vals_h.at[pl.ds(off, ch)], vs[b], sls[b]).wait()

                @pl.when(i >= 1)
                def _():
                    pltpu.make_async_copy(
                        vs[1 - b], acc_sh.at[ixs[1 - b]], scs[1 - b]).wait()

                pltpu.async_copy(vs[b], acc_sh.at[ixs[b]], scs[b], add=True)

                @pl.when(i < nit - 1)
                def _():
                    pltpu.sync_copy(dst_h.at[pl.ds(off + ch, ch)], ixs[1 - b])
                    pltpu.async_copy(
                        vals_h.at[pl.ds(off + ch, ch)], vs[1 - b], sls[1 - b])
            return carry

        lax.fori_loop(0, nit // 2, outer, 0)
        pltpu.make_async_copy(vs[1], acc_sh.at[ixs[1]], scs[1]).wait()
        plsc.subcore_barrier()
        pltpu.sync_copy(acc_sh.at[pl.ds(sid * NPS, NPS)],
                        out_h.at[cid, pl.ds(sid * NPS, NPS)])

    return k(vals, dst, zeros)


def _logits_body(xj_ref, xi_ref, ee_ref, et_ref, att_ref, p_ref):
    s = xj_ref[0] + xi_ref[0] + ee_ref[0].astype(f32)
    s = jnp.maximum(s, 0.2 * s)
    onehot = _onehot(et_ref)
    l16 = s @ att_ref[...]
    p16 = jnp.repeat(onehot, H, axis=1) * jnp.exp(l16)
    p_ref[0] = jnp.concatenate([p16, jnp.zeros((ET, C - R * H), f32)], axis=1)


def _make_logits(xj3, xi3, ee3, etc3, attm, te):
    return pl.pallas_call(
        _logits_body,
        grid=(te,),
        in_specs=[
            pl.BlockSpec((1, ET, HC), lambda i: (i, 0, 0)),
            pl.BlockSpec((1, ET, HC), lambda i: (i, 0, 0)),
            pl.BlockSpec((1, ET, HC), lambda i: (i, 0, 0)),
            pl.BlockSpec((1, 1, ET), lambda i: (i, 0, 0)),
            pl.BlockSpec((HC, R * H), lambda i: (0, 0)),
        ],
        out_specs=pl.BlockSpec((1, ET, C), lambda i: (i, 0, 0)),
        out_shape=jax.ShapeDtypeStruct((te, ET, C), f32),
    )(xj3, xi3, ee3, etc3, attm)


def _deninv_body(denp_ref, denq_ref, gate_ref, di_ref):
    den = (denp_ref[0] + denp_ref[1]
           + denq_ref[0] + denq_ref[1])[:, :R * H]
    g = jax.nn.softmax(gate_ref[...], axis=-1)
    gf = jnp.repeat(g, H, axis=1) / H
    di = gf / (den + 1e-16)
    di_ref[...] = jnp.concatenate([di, jnp.zeros((NPS, C - R * H), f32)], axis=1)


def _make_deninv(denp, denq, gate2):
    return pl.pallas_call(
        _deninv_body,
        grid=(NS,),
        in_specs=[
            pl.BlockSpec((NC, NPS, C), lambda t: (0, t, 0)),
            pl.BlockSpec((NC, NPS, C), lambda t: (0, t, 0)),
            pl.BlockSpec((1, R), lambda t: (0, 0)),
        ],
        out_specs=pl.BlockSpec((NPS, C), lambda t: (t, 0)),
        out_shape=jax.ShapeDtypeStruct((NP, C), f32),
    )(denp, denq, gate2)


def _msg_body(p_ref, di_ref, xj_ref, v_ref):
    w16 = p_ref[0][:, :R * H] * di_ref[0][:, :R * H]
    ii = lax.broadcasted_iota(i32, (R * H, H), 0)
    jj = lax.broadcasted_iota(i32, (R * H, H), 1)
    sel = (ii % H == jj).astype(f32)
    w4 = w16 @ sel
    xj = xj_ref[0]
    acc = w4[:, 0:1] * xj[:, 0:C]
    for hh in range(1, H):
        acc = acc + w4[:, hh:hh + 1] * xj[:, hh * C:(hh + 1) * C]
    v_ref[0] = acc


def _make_msgs(p3, di3, xj3, te):
    return pl.pallas_call(
        _msg_body,
        grid=(te,),
        in_specs=[
            pl.BlockSpec((1, ET, C), lambda i: (i, 0, 0)),
            pl.BlockSpec((1, ET, C), lambda i: (i, 0, 0)),
            pl.BlockSpec((1, ET, HC), lambda i: (i, 0, 0)),
        ],
        out_specs=pl.BlockSpec((1, ET, C), lambda i: (i, 0, 0)),
        out_shape=jax.ShapeDtypeStruct((te, ET, C), f32),
    )(p3, di3, xj3)


def _final_body(h_ref, op_ref, oq_ref, gate_ref, cb_ref, l1w_ref, l1b_ref,
                l2w_ref, l2b_ref, w1_ref, b1_ref, w2_ref, b2_ref, o_ref):
    g = jax.nn.softmax(gate_ref[...], axis=-1)
    const = g @ cb_ref[...]
    x = (h_ref[...] + op_ref[0] + op_ref[1]
         + oq_ref[0] + oq_ref[1] + const)
    mu = jnp.mean(x, axis=-1, keepdims=True)
    var = jnp.mean((x - mu) ** 2, axis=-1, keepdims=True)
    h1 = (x - mu) / jnp.sqrt(var + 1e-5) * l1w_ref[...] + l1b_ref[...]
    t = h1 @ w1_ref[...] + b1_ref[...]
    t = t * jax.nn.sigmoid(t)
    y = t @ w2_ref[...] + b2_ref[...]
    x2 = h1 + y
    mu2 = jnp.mean(x2, axis=-1, keepdims=True)
    var2 = jnp.mean((x2 - mu2) ** 2, axis=-1, keepdims=True)
    o_ref[...] = (x2 - mu2) / jnp.sqrt(var2 + 1e-5) * l2w_ref[...] + l2b_ref[...]


def _make_final(h, outp, outq, gate2, conv_bias, ln1w2, ln1b2, ln2w2, ln2b2,
                ffn_w1, ffn_b12, ffn_w2, ffn_b22):
    return pl.pallas_call(
        _final_body,
        grid=(TN,),
        in_specs=[
            pl.BlockSpec((NT, D), lambda t: (t, 0)),
            pl.BlockSpec((NC, NT, C), lambda t: (0, t, 0)),
            pl.BlockSpec((NC, NT, C), lambda t: (0, t, 0)),
            pl.BlockSpec((1, R), lambda t: (0, 0)),
            pl.BlockSpec((R, C), lambda t: (0, 0)),
            pl.BlockSpec((1, D), lambda t: (0, 0)),
            pl.BlockSpec((1, D), lambda t: (0, 0)),
            pl.BlockSpec((1, D), lambda t: (0, 0)),
            pl.BlockSpec((1, D), lambda t: (0, 0)),
            pl.BlockSpec((D, FFN), lambda t: (0, 0)),
            pl.BlockSpec((1, FFN), lambda t: (0, 0)),
            pl.BlockSpec((FFN, D), lambda t: (0, 0)),
            pl.BlockSpec((1, D), lambda t: (0, 0)),
        ],
        out_specs=pl.BlockSpec((NT, D), lambda t: (t, 0)),
        out_shape=jax.ShapeDtypeStruct((N, D), f32),
    )(h, outp, outq, gate2, conv_bias, ln1w2, ln1b2, ln2w2, ln2b2,
      ffn_w1, ffn_b12, ffn_w2, ffn_b22)


def kernel(h, edge_index, edge_attr, edge_type, rel_emb, rel_gate, W_l, b_l,
           W_r, b_r, W_e, att, conv_bias, ln1_w, ln1_b, ln2_w, ln2_b,
           ffn_w1, ffn_b1, ffn_w2, ffn_b2):
    src = edge_index[0]
    dst = edge_index[1]
    gate2 = rel_gate.reshape(1, R)
    attm = jnp.einsum('rhc,hg->hcrg', att, jnp.eye(H, dtype=f32))
    attm = attm.reshape(HC, R * H)

    XL, XR = _make_tables(h, W_l, b_l, W_r, b_r, W_e, rel_emb)
    xlf = XL.reshape(R * N, HC)
    xrf = XR.reshape(R * N, HC)

    et3s, dsts, ee3s, xjs, p3s = [], [], [], [], []
    for (hb, ne, te) in HALVES:
        ew = ne // NW
        et3 = lax.dynamic_slice_in_dim(edge_type, hb, ne).reshape(te, 1, ET)
        src3 = lax.dynamic_slice_in_dim(src, hb, ne).reshape(te, 1, ET)
        dst3 = lax.dynamic_slice_in_dim(dst, hb, ne).reshape(te, 1, ET)
        dsth = lax.dynamic_slice_in_dim(dst, hb, ne)
        ea3 = lax.dynamic_slice_in_dim(edge_attr, hb, ne).reshape(te, ET, EA)
        idxl3, idxr3 = _make_idx(src3, dst3, et3, te)
        XJ, XI = _sc_gather2(xlf, xrf, idxl3.reshape(ne), idxr3.reshape(ne),
                             ne, ew, ew // CH)
        EE3 = _make_ee(ea3, et3, W_e, te)
        P3 = _make_logits(XJ.reshape(te, ET, HC), XI.reshape(te, ET, HC),
                          EE3, et3, attm, te)
        et3s.append(et3)
        dsts.append(dsth)
        xjs.append(XJ)
        p3s.append(P3)

    denps = [
        _sc_scatter_add(p3s[k].reshape(ne, C), dsts[k],
                        jnp.zeros((NP, C), f32), C, ne, ne // NW,
                        ne // NW // _chs(ne), _chs(ne))
        for k, (hb, ne, te) in enumerate(HALVES)
    ]
    DENINV = _make_deninv(denps[0], denps[1], gate2)

    outps = []
    for k, (hb, ne, te) in enumerate(HALVES):
        ew = ne // NW
        DI = _sc_gather1(DENINV, dsts[k], C, ne, ew, ew // _chs(ne), _chs(ne))
        V3 = _make_msgs(p3s[k], DI.reshape(te, ET, C),
                        xjs[k].reshape(te, ET, HC), te)
        outps.append(_sc_scatter_add(V3.reshape(ne, C), dsts[k],
                                     jnp.zeros((NP, C), f32), C,
                                     ne, ew, ew // _chs(ne), _chs(ne)))

    return _make_final(h, outps[0][:, :N, :], outps[1][:, :N, :], gate2,
                       conv_bias,
                       ln1_w.reshape(1, D), ln1_b.reshape(1, D),
                       ln2_w.reshape(1, D), ln2_b.reshape(1, D),
                       ffn_w1, ffn_b1.reshape(1, FFN),
                       ffn_w2, ffn_b2.reshape(1, D))
